# Initial kernel scaffold; baseline (speedup 1.0000x reference)
import functools

import jax
import jax.numpy as jnp
from jax import lax
from jax.experimental import pallas as pl
from jax.experimental.pallas import tpu as pltpu
from jax.experimental.pallas import tpu_sc as plsc

N = 10000
E = 320000
D = 128
NP = 10240
NC = 2
NS = 16
NW = NC * NS
EPW = E // NW
K = 80
NCH = EPW // K
RPW = NP // NW
RPS = NP // NS

_mesh = plsc.VectorSubcoreMesh(core_axis_name="c", subcore_axis_name="s")


def _floop(n, body):
  lax.fori_loop(0, n, lambda i, c: (body(i), c)[1], 0)


def _bcast16(val_ref, i):
  return plsc.load_gather(val_ref, [jnp.full((16,), i, jnp.int32)])


def _tc_prep_body(x_ref, w1_ref, w2_ref, b1_ref, u_ref, v_ref, c_ref, c0_ref):
  W2 = w2_ref[...]
  xv = x_ref[...]
  A = jnp.dot(w1_ref[0:D, :], W2, preferred_element_type=jnp.float32)
  B = jnp.dot(w1_ref[D:2 * D, :], W2, preferred_element_type=jnp.float32)
  u_ref[...] = jnp.dot(xv, A, preferred_element_type=jnp.float32)
  v_ref[...] = jnp.dot(xv, B, preferred_element_type=jnp.float32)
  c_ref[...] = jnp.dot(w1_ref[2 * D:3 * D, :], W2,
                       preferred_element_type=jnp.float32)
  c0_ref[...] = jnp.dot(b1_ref[...], W2, preferred_element_type=jnp.float32)


def _tc_prep(xp, W1, W2, b1row):
  return pl.pallas_call(
      _tc_prep_body,
      out_shape=[
          jax.ShapeDtypeStruct((NP, D), jnp.float32),
          jax.ShapeDtypeStruct((NP, D), jnp.float32),
          jax.ShapeDtypeStruct((D, D), jnp.float32),
          jax.ShapeDtypeStruct((1, D), jnp.float32),
      ],
  )(xp, W1, W2, b1row)


_MBLK = 2000


def _tc_mm_body(ea_ref, c_ref, c0_ref, o_ref):
  o_ref[...] = jnp.dot(ea_ref[...], c_ref[...],
                       preferred_element_type=jnp.float32) + c0_ref[...]


def _tc_mm(edge_attr, Cm, c0):
  return pl.pallas_call(
      _tc_mm_body,
      grid=(E // _MBLK,),
      in_specs=[
          pl.BlockSpec((_MBLK, D), lambda i: (i, 0)),
          pl.BlockSpec((D, D), lambda i: (0, 0)),
          pl.BlockSpec((1, D), lambda i: (0, 0)),
      ],
      out_specs=pl.BlockSpec((_MBLK, D), lambda i: (i, 0)),
      out_shape=jax.ShapeDtypeStruct((E, D), jnp.float32),
  )(edge_attr, Cm, c0)


def _tc_deg_body(p_ref, dinv_ref, dinv2_ref):
  s = p_ref[0, :] + p_ref[1, :] + 1.0
  dinv_ref[...] = lax.rsqrt(s)
  dinv2_ref[...] = 1.0 / s


def _tc_deg(degp):
  return pl.pallas_call(
      _tc_deg_body,
      out_shape=[
          jax.ShapeDtypeStruct((NP,), jnp.float32),
          jax.ShapeDtypeStruct((NP,), jnp.float32),
      ],
  )(degp)


@functools.partial(
    pl.kernel,
    out_type=jax.ShapeDtypeStruct((NC, NP), jnp.float32),
    mesh=_mesh,
    scratch_types=[
        pltpu.VMEM((K,), jnp.int32),
        pltpu.VMEM((K,), jnp.float32),
        pltpu.VMEM((RPS,), jnp.float32),
        pltpu.VMEM_SHARED((NP,), jnp.float32),
    ],
)
def _sc_indeg(dst_hbm, out_hbm, idx_v, ones_v, zb, deg_sh):
  cid = lax.axis_index("c")
  sid = lax.axis_index("s")
  wid = cid * NS + sid

  def zset(i):
    zb[pl.ds(i * 16, 16)] = jnp.zeros((16,), jnp.float32)
  _floop(RPS // 16, zset)

  def oset(i):
    ones_v[pl.ds(i * 16, 16)] = jnp.ones((16,), jnp.float32)
  _floop(K // 16, oset)

  pltpu.sync_copy(zb, deg_sh.at[pl.ds(sid * RPS, RPS)])
  plsc.subcore_barrier()

  def chunk(j):
    base = wid * EPW + j * K
    pltpu.sync_copy(dst_hbm.at[pl.ds(base, K)], idx_v)
    pltpu.sync_copy(ones_v, deg_sh.at[idx_v], add=True)
  _floop(NCH, chunk)

  plsc.subcore_barrier()
  pltpu.sync_copy(deg_sh.at[pl.ds(sid * RPS, RPS)],
                  out_hbm.at[cid, pl.ds(sid * RPS, RPS)])


@functools.partial(
    pl.kernel,
    out_type=jax.ShapeDtypeStruct((NP, D), jnp.float32),
    mesh=_mesh,
    scratch_types=[
        pltpu.VMEM((RPW,), jnp.float32),
        pltpu.VMEM((K, D), jnp.float32),
    ],
)
def _sc_scale1(u_hbm, dinv_hbm, g1_hbm, dv, ub):
  cid = lax.axis_index("c")
  sid = lax.axis_index("s")
  wid = cid * NS + sid
  r0 = wid * RPW
  pltpu.sync_copy(dinv_hbm.at[pl.ds(r0, RPW)], dv)
  for c in range(RPW // K):
    pltpu.sync_copy(u_hbm.at[pl.ds(r0 + c * K, K), :], ub)

    def row(i):
      db = _bcast16(dv, c * K + i)
      for f in range(D // 16):
        sl = pl.ds(f * 16, 16)
        ub[i, sl] = ub[i, sl] * db
    _floop(K, row)
    pltpu.sync_copy(ub, g1_hbm.at[pl.ds(r0 + c * K, K), :])


@functools.partial(
    pl.kernel,
    out_type=[
        jax.ShapeDtypeStruct((E, D), jnp.float32),
        jax.ShapeDtypeStruct((NC, NP, D), jnp.float32),
    ],
    mesh=_mesh,
    scratch_types=[
        pltpu.VMEM((NP,), jnp.float32),
        pltpu.VMEM((NP,), jnp.float32),
        pltpu.VMEM((D,), jnp.float32),
        pltpu.VMEM((K,), jnp.int32),
        pltpu.VMEM((K,), jnp.int32),
        pltpu.VMEM((K,), jnp.float32),
        pltpu.VMEM((K,), jnp.float32),
        pltpu.VMEM((K, D), jnp.float32),
        pltpu.VMEM((K, D), jnp.float32),
        pltpu.VMEM((K, D), jnp.float32),
        pltpu.VMEM((K, D), jnp.float32),
        pltpu.VMEM((K, D), jnp.float32),
        pltpu.VMEM_SHARED((NP, D), jnp.float32),
    ],
)
def _sc_main(ea_hbm, src_hbm, dst_hbm, v_hbm, g1_hbm, dinv_hbm, dinv2_hbm,
             b2_hbm, p_hbm, tpart_hbm,
             dv, dv2, b2v, src_v, dst_v, w_v, w2_v, sb, vb, gb, yb, pb, tsh):
  cid = lax.axis_index("c")
  sid = lax.axis_index("s")
  wid = cid * NS + sid

  pltpu.sync_copy(dinv_hbm, dv)
  pltpu.sync_copy(dinv2_hbm, dv2)
  pltpu.sync_copy(b2_hbm, b2v)

  def zrow(i):
    for f in range(D // 16):
      yb[i, pl.ds(f * 16, 16)] = jnp.zeros((16,), jnp.float32)
  _floop(K, zrow)
  for q in range(RPS // K):
    pltpu.sync_copy(yb, tsh.at[pl.ds(sid * RPS + q * K, K), :])
  plsc.subcore_barrier()

  def chunk(j):
    base = wid * EPW + j * K
    pltpu.sync_copy(src_hbm.at[pl.ds(base, K)], src_v)
    pltpu.sync_copy(dst_hbm.at[pl.ds(base, K)], dst_v)
    pltpu.sync_copy(ea_hbm.at[pl.ds(base, K), :], sb)
    pltpu.sync_copy(v_hbm.at[dst_v], vb)
    pltpu.sync_copy(g1_hbm.at[src_v], gb)
    for q in range(K // 16):
      sl = pl.ds(q * 16, 16)
      idx = src_v[sl]
      w_v[sl] = plsc.load_gather(dv, [idx])
      w2_v[sl] = plsc.load_gather(dv2, [idx])

    def row(i):
      wb = _bcast16(w_v, i)
      w2b = _bcast16(w2_v, i)
      for f in range(D // 16):
        sl = pl.ds(f * 16, 16)
        s = sb[i, sl] + vb[i, sl]
        yb[i, sl] = s * wb + gb[i, sl]
        pb[i, sl] = s * w2b + b2v[sl]
    _floop(K, row)

    pltpu.sync_copy(yb, tsh.at[dst_v], add=True)
    pltpu.sync_copy(pb, p_hbm.at[pl.ds(base, K), :])
  _floop(NCH, chunk)

  plsc.subcore_barrier()
  pltpu.sync_copy(tsh.at[pl.ds(sid * RPS, RPS), :],
                  tpart_hbm.at[cid, pl.ds(sid * RPS, RPS), :])


@functools.partial(
    pl.kernel,
    out_type=jax.ShapeDtypeStruct((NP, D), jnp.float32),
    mesh=_mesh,
    scratch_types=[
        pltpu.VMEM((RPW,), jnp.float32),
        pltpu.VMEM((RPW,), jnp.float32),
        pltpu.VMEM((K, D), jnp.float32),
        pltpu.VMEM((K, D), jnp.float32),
        pltpu.VMEM((K, D), jnp.float32),
    ],
)
def _sc_scale2(tpart_hbm, u_hbm, dinv_hbm, dinv2_hbm, g_hbm,
               dv, dv2, t0, t1, ub):
  cid = lax.axis_index("c")
  sid = lax.axis_index("s")
  wid = cid * NS + sid
  r0 = wid * RPW
  pltpu.sync_copy(dinv_hbm.at[pl.ds(r0, RPW)], dv)
  pltpu.sync_copy(dinv2_hbm.at[pl.ds(r0, RPW)], dv2)
  for c in range(RPW // K):
    rb = r0 + c * K
    pltpu.sync_copy(tpart_hbm.at[0, pl.ds(rb, K), :], t0)
    pltpu.sync_copy(tpart_hbm.at[1, pl.ds(rb, K), :], t1)
    pltpu.sync_copy(u_hbm.at[pl.ds(rb, K), :], ub)

    def row(i):
      db = _bcast16(dv, c * K + i)
      d2b = _bcast16(dv2, c * K + i)
      for f in range(D // 16):
        sl = pl.ds(f * 16, 16)
        t0[i, sl] = (t0[i, sl] + t1[i, sl]) * db + ub[i, sl] * d2b
    _floop(K, row)
    pltpu.sync_copy(t0, g_hbm.at[pl.ds(rb, K), :])


@functools.partial(
    pl.kernel,
    out_type=jax.ShapeDtypeStruct((E, D), jnp.float32),
    mesh=_mesh,
    scratch_types=[
        pltpu.VMEM((K,), jnp.int32),
        pltpu.VMEM((K, D), jnp.float32),
        pltpu.VMEM((K, D), jnp.float32),
    ],
)
def _sc_out(p_hbm, src_hbm, g_hbm, out_hbm, src_v, ob, gb):
  cid = lax.axis_index("c")
  sid = lax.axis_index("s")
  wid = cid * NS + sid

  def chunk(j):
    base = wid * EPW + j * K
    pltpu.sync_copy(src_hbm.at[pl.ds(base, K)], src_v)
    pltpu.sync_copy(p_hbm.at[pl.ds(base, K), :], ob)
    pltpu.sync_copy(g_hbm.at[src_v], gb)

    def row(i):
      for f in range(D // 16):
        sl = pl.ds(f * 16, 16)
        o = ob[i, sl] + gb[i, sl]
        ob[i, sl] = jnp.maximum(o, 0.01 * o)
    _floop(K, row)
    pltpu.sync_copy(ob, out_hbm.at[pl.ds(base, K), :])
  _floop(NCH, chunk)


@jax.jit
def kernel(x, edge_index, batch, edge_attr, W1, b1, W2, b2):
  src = edge_index[0]
  dst = edge_index[1]
  xp = jnp.pad(x, ((0, NP - N), (0, 0)))
  u, v, Cm, c0 = _tc_prep(xp, W1, W2, b1.reshape(1, D))
  ea = _tc_mm(edge_attr, Cm, c0)
  degp = _sc_indeg(dst)
  dinv, dinv2 = _tc_deg(degp)
  g1 = _sc_scale1(u, dinv)
  p, tpart = _sc_main(ea, src, dst, v, g1, dinv, dinv2, b2)
  G = _sc_scale2(tpart, u, dinv, dinv2)
  return _sc_out(p, src, G)


# speedup vs baseline: 2.3338x; 2.3338x over previous
import functools

import jax
import jax.numpy as jnp
from jax import lax
from jax.experimental import pallas as pl
from jax.experimental.pallas import tpu as pltpu
from jax.experimental.pallas import tpu_sc as plsc

N = 10000
E = 320000
D = 128
NP = 10240
NC = 2
NS = 16
NW = NC * NS
EPW = E // NW
K = 80
NCH = EPW // K
RPW = NP // NW
RPS = NP // NS

_mesh = plsc.VectorSubcoreMesh(core_axis_name="c", subcore_axis_name="s")
_sc_params = pltpu.CompilerParams(needs_layout_passes=False)


def _floop(n, body):
  lax.fori_loop(0, n, lambda i, c: (body(i), c)[1], 0)


def _bcast16(val_ref, i):
  return plsc.load_gather(val_ref, [jnp.full((16,), i, jnp.int32)])


def _tc_prep_body(x_ref, w1_ref, w2_ref, b1_ref, u_ref, v_ref, c_ref, c0_ref):
  W2 = w2_ref[...]
  xv = x_ref[...]
  A = jnp.dot(w1_ref[0:D, :], W2, preferred_element_type=jnp.float32)
  B = jnp.dot(w1_ref[D:2 * D, :], W2, preferred_element_type=jnp.float32)
  u_ref[...] = jnp.dot(xv, A, preferred_element_type=jnp.float32)
  v_ref[...] = jnp.dot(xv, B, preferred_element_type=jnp.float32)
  c_ref[...] = jnp.dot(w1_ref[2 * D:3 * D, :], W2,
                       preferred_element_type=jnp.float32)
  c0_ref[...] = jnp.dot(b1_ref[...], W2, preferred_element_type=jnp.float32)


def _tc_prep(xp, W1, W2, b1row):
  return pl.pallas_call(
      _tc_prep_body,
      out_shape=[
          jax.ShapeDtypeStruct((NP, D), jnp.float32),
          jax.ShapeDtypeStruct((NP, D), jnp.float32),
          jax.ShapeDtypeStruct((D, D), jnp.float32),
          jax.ShapeDtypeStruct((1, D), jnp.float32),
      ],
  )(xp, W1, W2, b1row)


_MBLK = 2000


def _tc_mm_body(ea_ref, c_ref, c0_ref, o_ref):
  o_ref[...] = jnp.dot(ea_ref[...], c_ref[...],
                       preferred_element_type=jnp.float32) + c0_ref[...]


def _tc_mm(edge_attr, Cm, c0):
  return pl.pallas_call(
      _tc_mm_body,
      grid=(E // _MBLK,),
      in_specs=[
          pl.BlockSpec((_MBLK, D), lambda i: (i, 0)),
          pl.BlockSpec((D, D), lambda i: (0, 0)),
          pl.BlockSpec((1, D), lambda i: (0, 0)),
      ],
      out_specs=pl.BlockSpec((_MBLK, D), lambda i: (i, 0)),
      out_shape=jax.ShapeDtypeStruct((E, D), jnp.float32),
  )(edge_attr, Cm, c0)


def _tc_deg_body(p_ref, dinv_ref, dinv2_ref):
  s = p_ref[0, :] + p_ref[1, :] + 1.0
  dinv_ref[...] = lax.rsqrt(s)
  dinv2_ref[...] = 1.0 / s


def _tc_deg(degp):
  return pl.pallas_call(
      _tc_deg_body,
      out_shape=[
          jax.ShapeDtypeStruct((NP,), jnp.float32),
          jax.ShapeDtypeStruct((NP,), jnp.float32),
      ],
  )(degp)


@functools.partial(
    pl.kernel,
    out_type=jax.ShapeDtypeStruct((NC, NP), jnp.float32),
    mesh=_mesh,
    compiler_params=_sc_params,
    scratch_types=[
        pltpu.VMEM((K,), jnp.int32),
        pltpu.VMEM((K,), jnp.float32),
        pltpu.VMEM((RPS,), jnp.float32),
        pltpu.VMEM_SHARED((NP,), jnp.float32),
    ],
)
def _sc_indeg(dst_hbm, out_hbm, idx_v, ones_v, zb, deg_sh):
  cid = lax.axis_index("c")
  sid = lax.axis_index("s")
  wid = cid * NS + sid

  def zset(i):
    zb[pl.ds(i * 16, 16)] = jnp.zeros((16,), jnp.float32)
  _floop(RPS // 16, zset)

  def oset(i):
    ones_v[pl.ds(i * 16, 16)] = jnp.ones((16,), jnp.float32)
  _floop(K // 16, oset)

  pltpu.sync_copy(zb, deg_sh.at[pl.ds(sid * RPS, RPS)])
  plsc.subcore_barrier()

  def chunk(j):
    base = wid * EPW + j * K
    pltpu.sync_copy(dst_hbm.at[pl.ds(base, K)], idx_v)
    pltpu.sync_copy(ones_v, deg_sh.at[idx_v], add=True)
  _floop(NCH, chunk)

  plsc.subcore_barrier()
  pltpu.sync_copy(deg_sh.at[pl.ds(sid * RPS, RPS)],
                  out_hbm.at[cid, pl.ds(sid * RPS, RPS)])


@functools.partial(
    pl.kernel,
    out_type=jax.ShapeDtypeStruct((NP, D), jnp.float32),
    mesh=_mesh,
    compiler_params=_sc_params,
    scratch_types=[
        pltpu.VMEM((RPW,), jnp.float32),
        pltpu.VMEM((K, D), jnp.float32),
    ],
)
def _sc_scale1(u_hbm, dinv_hbm, g1_hbm, dv, ub):
  cid = lax.axis_index("c")
  sid = lax.axis_index("s")
  wid = cid * NS + sid
  r0 = wid * RPW
  pltpu.sync_copy(dinv_hbm.at[pl.ds(r0, RPW)], dv)
  for c in range(RPW // K):
    pltpu.sync_copy(u_hbm.at[pl.ds(r0 + c * K, K), :], ub)

    def row(i):
      db = _bcast16(dv, c * K + i)
      for f in range(D // 16):
        sl = pl.ds(f * 16, 16)
        ub[i, sl] = ub[i, sl] * db
    _floop(K, row)
    pltpu.sync_copy(ub, g1_hbm.at[pl.ds(r0 + c * K, K), :])


@functools.partial(
    pl.kernel,
    out_type=[
        jax.ShapeDtypeStruct((E, D), jnp.float32),
        jax.ShapeDtypeStruct((NC, NP, D), jnp.float32),
    ],
    mesh=_mesh,
    compiler_params=_sc_params,
    scratch_types=[
        pltpu.VMEM((D,), jnp.float32),
        pltpu.VMEM((K,), jnp.int32),
        pltpu.VMEM((K,), jnp.int32),
        pltpu.VMEM((K,), jnp.float32),
        pltpu.VMEM((K,), jnp.float32),
        pltpu.VMEM((K, D), jnp.float32),
        pltpu.VMEM((K, D), jnp.float32),
        pltpu.VMEM((K, D), jnp.float32),
        pltpu.VMEM_SHARED((NP, D), jnp.float32),
    ],
)
def _sc_main(ea_hbm, src_hbm, dst_hbm, v_hbm, g1_hbm, dinv_hbm, dinv2_hbm,
             b2_hbm, p_hbm, tpart_hbm,
             b2v, src_v, dst_v, w_v, w2_v, sb, vb, gb, tsh):
  cid = lax.axis_index("c")
  sid = lax.axis_index("s")
  wid = cid * NS + sid

  pltpu.sync_copy(b2_hbm, b2v)

  def zrow(i):
    for f in range(D // 16):
      vb[i, pl.ds(f * 16, 16)] = jnp.zeros((16,), jnp.float32)
  _floop(K, zrow)
  for q in range(RPS // K):
    pltpu.sync_copy(vb, tsh.at[pl.ds(sid * RPS + q * K, K), :])
  plsc.subcore_barrier()

  def chunk(j):
    base = wid * EPW + j * K
    pltpu.sync_copy(src_hbm.at[pl.ds(base, K)], src_v)
    pltpu.sync_copy(dst_hbm.at[pl.ds(base, K)], dst_v)
    pltpu.sync_copy(ea_hbm.at[pl.ds(base, K), :], sb)
    pltpu.sync_copy(v_hbm.at[dst_v], vb)
    pltpu.sync_copy(g1_hbm.at[src_v], gb)
    pltpu.sync_copy(dinv_hbm.at[src_v], w_v)
    pltpu.sync_copy(dinv2_hbm.at[src_v], w2_v)

    def row(i):
      wb = _bcast16(w_v, i)
      w2b = _bcast16(w2_v, i)
      for f in range(D // 16):
        sl = pl.ds(f * 16, 16)
        s = sb[i, sl] + vb[i, sl]
        vb[i, sl] = s * wb + gb[i, sl]
        sb[i, sl] = s * w2b + b2v[sl]
    _floop(K, row)

    pltpu.sync_copy(vb, tsh.at[dst_v], add=True)
    pltpu.sync_copy(sb, p_hbm.at[pl.ds(base, K), :])
  _floop(NCH, chunk)

  plsc.subcore_barrier()
  pltpu.sync_copy(tsh.at[pl.ds(sid * RPS, RPS), :],
                  tpart_hbm.at[cid, pl.ds(sid * RPS, RPS), :])


@functools.partial(
    pl.kernel,
    out_type=jax.ShapeDtypeStruct((NP, D), jnp.float32),
    mesh=_mesh,
    compiler_params=_sc_params,
    scratch_types=[
        pltpu.VMEM((RPW,), jnp.float32),
        pltpu.VMEM((RPW,), jnp.float32),
        pltpu.VMEM((K, D), jnp.float32),
        pltpu.VMEM((K, D), jnp.float32),
        pltpu.VMEM((K, D), jnp.float32),
    ],
)
def _sc_scale2(tpart_hbm, u_hbm, dinv_hbm, dinv2_hbm, g_hbm,
               dv, dv2, t0, t1, ub):
  cid = lax.axis_index("c")
  sid = lax.axis_index("s")
  wid = cid * NS + sid
  r0 = wid * RPW
  pltpu.sync_copy(dinv_hbm.at[pl.ds(r0, RPW)], dv)
  pltpu.sync_copy(dinv2_hbm.at[pl.ds(r0, RPW)], dv2)
  for c in range(RPW // K):
    rb = r0 + c * K
    pltpu.sync_copy(tpart_hbm.at[0, pl.ds(rb, K), :], t0)
    pltpu.sync_copy(tpart_hbm.at[1, pl.ds(rb, K), :], t1)
    pltpu.sync_copy(u_hbm.at[pl.ds(rb, K), :], ub)

    def row(i):
      db = _bcast16(dv, c * K + i)
      d2b = _bcast16(dv2, c * K + i)
      for f in range(D // 16):
        sl = pl.ds(f * 16, 16)
        t0[i, sl] = (t0[i, sl] + t1[i, sl]) * db + ub[i, sl] * d2b
    _floop(K, row)
    pltpu.sync_copy(t0, g_hbm.at[pl.ds(rb, K), :])


@functools.partial(
    pl.kernel,
    out_type=jax.ShapeDtypeStruct((E, D), jnp.float32),
    mesh=_mesh,
    compiler_params=_sc_params,
    scratch_types=[
        pltpu.VMEM((K,), jnp.int32),
        pltpu.VMEM((K, D), jnp.float32),
        pltpu.VMEM((K, D), jnp.float32),
    ],
)
def _sc_out(p_hbm, src_hbm, g_hbm, out_hbm, src_v, ob, gb):
  cid = lax.axis_index("c")
  sid = lax.axis_index("s")
  wid = cid * NS + sid

  def chunk(j):
    base = wid * EPW + j * K
    pltpu.sync_copy(src_hbm.at[pl.ds(base, K)], src_v)
    pltpu.sync_copy(p_hbm.at[pl.ds(base, K), :], ob)
    pltpu.sync_copy(g_hbm.at[src_v], gb)

    def row(i):
      for f in range(D // 16):
        sl = pl.ds(f * 16, 16)
        o = ob[i, sl] + gb[i, sl]
        ob[i, sl] = jnp.maximum(o, 0.01 * o)
    _floop(K, row)
    pltpu.sync_copy(ob, out_hbm.at[pl.ds(base, K), :])
  _floop(NCH, chunk)


@jax.jit
def kernel(x, edge_index, batch, edge_attr, W1, b1, W2, b2):
  src = edge_index[0]
  dst = edge_index[1]
  xp = jnp.pad(x, ((0, NP - N), (0, 0)))
  u, v, Cm, c0 = _tc_prep(xp, W1, W2, b1.reshape(1, D))
  ea = _tc_mm(edge_attr, Cm, c0)
  degp = _sc_indeg(dst)
  dinv, dinv2 = _tc_deg(degp)
  g1 = _sc_scale1(u, dinv)
  p, tpart = _sc_main(ea, src, dst, v, g1, dinv, dinv2, b2)
  G = _sc_scale2(tpart, u, dinv, dinv2)
  return _sc_out(p, src, G)


# speedup vs baseline: 2.6005x; 1.1143x over previous
import functools

import jax
import jax.numpy as jnp
from jax import lax
from jax.experimental import pallas as pl
from jax.experimental.pallas import tpu as pltpu
from jax.experimental.pallas import tpu_sc as plsc

N = 10000
E = 320000
D = 128
NP = 10240
NC = 2
NS = 16
NW = NC * NS
EPW = E // NW
K = 80
NCH = EPW // K
RPW = NP // NW
RPS = NP // NS

_mesh = plsc.VectorSubcoreMesh(core_axis_name="c", subcore_axis_name="s")
_sc_params = pltpu.CompilerParams(needs_layout_passes=False)


def _floop(n, body):
  lax.fori_loop(0, n, lambda i, c: (body(i), c)[1], 0)


def _bcast16(val_ref, i):
  return plsc.load_gather(val_ref, [jnp.full((16,), i, jnp.int32)])


def _tc_prep_body(x_ref, w1_ref, w2_ref, b1_ref, u_ref, v_ref, c_ref, c0_ref):
  W2 = w2_ref[...]
  xv = x_ref[...]
  A = jnp.dot(w1_ref[0:D, :], W2, preferred_element_type=jnp.float32)
  B = jnp.dot(w1_ref[D:2 * D, :], W2, preferred_element_type=jnp.float32)
  u_ref[...] = jnp.dot(xv, A, preferred_element_type=jnp.float32)
  v_ref[...] = jnp.dot(xv, B, preferred_element_type=jnp.float32)
  c_ref[...] = jnp.dot(w1_ref[2 * D:3 * D, :], W2,
                       preferred_element_type=jnp.float32)
  c0_ref[...] = jnp.dot(b1_ref[...], W2, preferred_element_type=jnp.float32)


def _tc_prep(xp, W1, W2, b1row):
  return pl.pallas_call(
      _tc_prep_body,
      out_shape=[
          jax.ShapeDtypeStruct((NP, D), jnp.float32),
          jax.ShapeDtypeStruct((NP, D), jnp.float32),
          jax.ShapeDtypeStruct((D, D), jnp.float32),
          jax.ShapeDtypeStruct((1, D), jnp.float32),
      ],
  )(xp, W1, W2, b1row)


_MBLK = 2000


def _tc_mm_body(ea_ref, c_ref, c0_ref, o_ref):
  o_ref[...] = jnp.dot(ea_ref[...], c_ref[...],
                       preferred_element_type=jnp.float32) + c0_ref[...]


def _tc_mm(edge_attr, Cm, c0):
  return pl.pallas_call(
      _tc_mm_body,
      grid=(E // _MBLK,),
      in_specs=[
          pl.BlockSpec((_MBLK, D), lambda i: (i, 0)),
          pl.BlockSpec((D, D), lambda i: (0, 0)),
          pl.BlockSpec((1, D), lambda i: (0, 0)),
      ],
      out_specs=pl.BlockSpec((_MBLK, D), lambda i: (i, 0)),
      out_shape=jax.ShapeDtypeStruct((E, D), jnp.float32),
  )(edge_attr, Cm, c0)


def _tc_deg_body(p_ref, dinv_ref, dinv2_ref):
  s = p_ref[0, :] + p_ref[1, :] + 1.0
  dinv_ref[...] = lax.rsqrt(s)
  dinv2_ref[...] = 1.0 / s


def _tc_deg(degp):
  return pl.pallas_call(
      _tc_deg_body,
      out_shape=[
          jax.ShapeDtypeStruct((NP,), jnp.float32),
          jax.ShapeDtypeStruct((NP,), jnp.float32),
      ],
  )(degp)


@functools.partial(
    pl.kernel,
    out_type=jax.ShapeDtypeStruct((NC, NP), jnp.float32),
    mesh=_mesh,
    compiler_params=_sc_params,
    scratch_types=[
        pltpu.VMEM((K,), jnp.int32),
        pltpu.VMEM((K,), jnp.float32),
        pltpu.VMEM((RPS,), jnp.float32),
        pltpu.VMEM_SHARED((NP,), jnp.float32),
    ],
)
def _sc_indeg(dst_hbm, out_hbm, idx_v, ones_v, zb, deg_sh):
  cid = lax.axis_index("c")
  sid = lax.axis_index("s")
  wid = cid * NS + sid

  def zset(i):
    zb[pl.ds(i * 16, 16)] = jnp.zeros((16,), jnp.float32)
  _floop(RPS // 16, zset)

  def oset(i):
    ones_v[pl.ds(i * 16, 16)] = jnp.ones((16,), jnp.float32)
  _floop(K // 16, oset)

  pltpu.sync_copy(zb, deg_sh.at[pl.ds(sid * RPS, RPS)])
  plsc.subcore_barrier()

  def chunk(j):
    base = wid * EPW + j * K
    pltpu.sync_copy(dst_hbm.at[pl.ds(base, K)], idx_v)
    pltpu.sync_copy(ones_v, deg_sh.at[idx_v], add=True)
  _floop(NCH, chunk)

  plsc.subcore_barrier()
  pltpu.sync_copy(deg_sh.at[pl.ds(sid * RPS, RPS)],
                  out_hbm.at[cid, pl.ds(sid * RPS, RPS)])


@functools.partial(
    pl.kernel,
    out_type=jax.ShapeDtypeStruct((NP, D), jnp.float32),
    mesh=_mesh,
    compiler_params=_sc_params,
    scratch_types=[
        pltpu.VMEM((RPW,), jnp.float32),
        pltpu.VMEM((K, D), jnp.float32),
    ],
)
def _sc_scale1(u_hbm, dinv_hbm, g1_hbm, dv, ub):
  cid = lax.axis_index("c")
  sid = lax.axis_index("s")
  wid = cid * NS + sid
  r0 = wid * RPW
  pltpu.sync_copy(dinv_hbm.at[pl.ds(r0, RPW)], dv)
  for c in range(RPW // K):
    pltpu.sync_copy(u_hbm.at[pl.ds(r0 + c * K, K), :], ub)

    def row(i):
      db = _bcast16(dv, c * K + i)
      for f in range(D // 16):
        sl = pl.ds(f * 16, 16)
        ub[i, sl] = ub[i, sl] * db
    _floop(K, row)
    pltpu.sync_copy(ub, g1_hbm.at[pl.ds(r0 + c * K, K), :])


@functools.partial(
    pl.kernel,
    out_type=jax.ShapeDtypeStruct((NC, NP, D), jnp.float32),
    mesh=_mesh,
    compiler_params=_sc_params,
    scratch_types=[
        pltpu.VMEM((K,), jnp.int32),
        pltpu.VMEM((K,), jnp.int32),
        pltpu.VMEM((K,), jnp.float32),
        pltpu.VMEM((K, D), jnp.float32),
        pltpu.VMEM((K, D), jnp.float32),
        pltpu.VMEM((K, D), jnp.float32),
        pltpu.VMEM_SHARED((NP, D), jnp.float32),
        pltpu.SemaphoreType.DMA,
        pltpu.SemaphoreType.DMA,
        pltpu.SemaphoreType.DMA,
        pltpu.SemaphoreType.DMA,
    ],
)
def _sc_main(ea_hbm, src_hbm, dst_hbm, v_hbm, g1_hbm, dinv_hbm,
             tpart_hbm,
             src_v, dst_v, w_v, sb, vb, gb, tsh, sem0, sem1, sem2, sem3):
  cid = lax.axis_index("c")
  sid = lax.axis_index("s")
  wid = cid * NS + sid

  def zrow(i):
    for f in range(D // 16):
      vb[i, pl.ds(f * 16, 16)] = jnp.zeros((16,), jnp.float32)
  _floop(K, zrow)
  for q in range(RPS // K):
    pltpu.sync_copy(vb, tsh.at[pl.ds(sid * RPS + q * K, K), :])
  plsc.subcore_barrier()

  def chunk(j):
    base = wid * EPW + j * K
    c1 = pltpu.async_copy(src_hbm.at[pl.ds(base, K)], src_v, sem0)
    c2 = pltpu.async_copy(dst_hbm.at[pl.ds(base, K)], dst_v, sem1)
    c3 = pltpu.async_copy(ea_hbm.at[pl.ds(base, K), :], sb, sem2)
    c1.wait()
    c2.wait()
    c4 = pltpu.async_copy(v_hbm.at[dst_v], vb, sem0)
    c5 = pltpu.async_copy(g1_hbm.at[src_v], gb, sem1)
    c6 = pltpu.async_copy(dinv_hbm.at[src_v], w_v, sem3)
    c3.wait()
    c4.wait()
    c5.wait()
    c6.wait()

    def row(i):
      wb = _bcast16(w_v, i)
      for f in range(D // 16):
        sl = pl.ds(f * 16, 16)
        vb[i, sl] = (sb[i, sl] + vb[i, sl]) * wb + gb[i, sl]
    _floop(K, row)

    pltpu.sync_copy(vb, tsh.at[dst_v], add=True)
  _floop(NCH, chunk)

  plsc.subcore_barrier()
  pltpu.sync_copy(tsh.at[pl.ds(sid * RPS, RPS), :],
                  tpart_hbm.at[cid, pl.ds(sid * RPS, RPS), :])


@functools.partial(
    pl.kernel,
    out_type=jax.ShapeDtypeStruct((NP, D), jnp.float32),
    mesh=_mesh,
    compiler_params=_sc_params,
    scratch_types=[
        pltpu.VMEM((RPW,), jnp.float32),
        pltpu.VMEM((RPW,), jnp.float32),
        pltpu.VMEM((K, D), jnp.float32),
        pltpu.VMEM((K, D), jnp.float32),
        pltpu.VMEM((K, D), jnp.float32),
    ],
)
def _sc_scale2(tpart_hbm, u_hbm, dinv_hbm, dinv2_hbm, g_hbm,
               dv, dv2, t0, t1, ub):
  cid = lax.axis_index("c")
  sid = lax.axis_index("s")
  wid = cid * NS + sid
  r0 = wid * RPW
  pltpu.sync_copy(dinv_hbm.at[pl.ds(r0, RPW)], dv)
  pltpu.sync_copy(dinv2_hbm.at[pl.ds(r0, RPW)], dv2)
  for c in range(RPW // K):
    rb = r0 + c * K
    pltpu.sync_copy(tpart_hbm.at[0, pl.ds(rb, K), :], t0)
    pltpu.sync_copy(tpart_hbm.at[1, pl.ds(rb, K), :], t1)
    pltpu.sync_copy(u_hbm.at[pl.ds(rb, K), :], ub)

    def row(i):
      db = _bcast16(dv, c * K + i)
      d2b = _bcast16(dv2, c * K + i)
      for f in range(D // 16):
        sl = pl.ds(f * 16, 16)
        t0[i, sl] = (t0[i, sl] + t1[i, sl]) * db + ub[i, sl] * d2b
    _floop(K, row)
    pltpu.sync_copy(t0, g_hbm.at[pl.ds(rb, K), :])


@functools.partial(
    pl.kernel,
    out_type=jax.ShapeDtypeStruct((E, D), jnp.float32),
    mesh=_mesh,
    compiler_params=_sc_params,
    scratch_types=[
        pltpu.VMEM((K,), jnp.int32),
        pltpu.VMEM((K,), jnp.int32),
        pltpu.VMEM((K,), jnp.float32),
        pltpu.VMEM((D,), jnp.float32),
        pltpu.VMEM((K, D), jnp.float32),
        pltpu.VMEM((K, D), jnp.float32),
        pltpu.VMEM((K, D), jnp.float32),
        pltpu.SemaphoreType.DMA,
        pltpu.SemaphoreType.DMA,
        pltpu.SemaphoreType.DMA,
        pltpu.SemaphoreType.DMA,
    ],
)
def _sc_out(ea_hbm, src_hbm, dst_hbm, v_hbm, g_hbm, dinv2_hbm, b2_hbm,
            out_hbm, src_v, dst_v, w2_v, b2v, sb, vb, gb,
            sem0, sem1, sem2, sem3):
  cid = lax.axis_index("c")
  sid = lax.axis_index("s")
  wid = cid * NS + sid
  pltpu.sync_copy(b2_hbm, b2v)

  def chunk(j):
    base = wid * EPW + j * K
    c1 = pltpu.async_copy(src_hbm.at[pl.ds(base, K)], src_v, sem0)
    c2 = pltpu.async_copy(dst_hbm.at[pl.ds(base, K)], dst_v, sem1)
    c3 = pltpu.async_copy(ea_hbm.at[pl.ds(base, K), :], sb, sem2)
    c1.wait()
    c2.wait()
    c4 = pltpu.async_copy(v_hbm.at[dst_v], vb, sem0)
    c5 = pltpu.async_copy(g_hbm.at[src_v], gb, sem1)
    c6 = pltpu.async_copy(dinv2_hbm.at[src_v], w2_v, sem3)
    c3.wait()
    c4.wait()
    c5.wait()
    c6.wait()

    def row(i):
      w2b = _bcast16(w2_v, i)
      for f in range(D // 16):
        sl = pl.ds(f * 16, 16)
        o = (sb[i, sl] + vb[i, sl]) * w2b + gb[i, sl] + b2v[sl]
        sb[i, sl] = jnp.maximum(o, 0.01 * o)
    _floop(K, row)
    pltpu.sync_copy(sb, out_hbm.at[pl.ds(base, K), :])
  _floop(NCH, chunk)


@jax.jit
def kernel(x, edge_index, batch, edge_attr, W1, b1, W2, b2):
  src = edge_index[0]
  dst = edge_index[1]
  xp = jnp.pad(x, ((0, NP - N), (0, 0)))
  u, v, Cm, c0 = _tc_prep(xp, W1, W2, b1.reshape(1, D))
  ea = _tc_mm(edge_attr, Cm, c0)
  degp = _sc_indeg(dst)
  dinv, dinv2 = _tc_deg(degp)
  g1 = _sc_scale1(u, dinv)
  tpart = _sc_main(ea, src, dst, v, g1, dinv)
  G = _sc_scale2(tpart, u, dinv, dinv2)
  return _sc_out(ea, src, dst, v, G, dinv2, b2)


# speedup vs baseline: 2.9246x; 1.1246x over previous
import functools

import jax
import jax.numpy as jnp
from jax import lax
from jax.experimental import pallas as pl
from jax.experimental.pallas import tpu as pltpu
from jax.experimental.pallas import tpu_sc as plsc

N = 10000
E = 320000
D = 128
NP = 10240
NC = 2
NS = 16
NW = NC * NS
EPW = E // NW
K = 80
NCH = EPW // K
GRP = 5
GC = NCH // GRP
RPW = NP // NW
RPS = NP // NS

_mesh = plsc.VectorSubcoreMesh(core_axis_name="c", subcore_axis_name="s")
_sc_params = pltpu.CompilerParams(needs_layout_passes=False)


def _floop(n, body):
  lax.fori_loop(0, n, lambda i, c: (body(i), c)[1], 0)


def _bcast16(val_ref, i):
  return plsc.load_gather(val_ref, [jnp.full((16,), i, jnp.int32)])


def _tc_prep_body(x_ref, w1_ref, w2_ref, b1_ref, u_ref, v_ref, c_ref, c0_ref):
  W2 = w2_ref[...]
  xv = x_ref[...]
  A = jnp.dot(w1_ref[0:D, :], W2, preferred_element_type=jnp.float32)
  B = jnp.dot(w1_ref[D:2 * D, :], W2, preferred_element_type=jnp.float32)
  u_ref[...] = jnp.dot(xv, A, preferred_element_type=jnp.float32)
  v_ref[...] = jnp.dot(xv, B, preferred_element_type=jnp.float32)
  c_ref[...] = jnp.dot(w1_ref[2 * D:3 * D, :], W2,
                       preferred_element_type=jnp.float32)
  c0_ref[...] = jnp.dot(b1_ref[...], W2, preferred_element_type=jnp.float32)


def _tc_prep(xp, W1, W2, b1row):
  return pl.pallas_call(
      _tc_prep_body,
      out_shape=[
          jax.ShapeDtypeStruct((NP, D), jnp.float32),
          jax.ShapeDtypeStruct((NP, D), jnp.float32),
          jax.ShapeDtypeStruct((D, D), jnp.float32),
          jax.ShapeDtypeStruct((1, D), jnp.float32),
      ],
  )(xp, W1, W2, b1row)


_MBLK = 2000


def _tc_mm_body(ea_ref, c_ref, c0_ref, o_ref):
  o_ref[...] = jnp.dot(ea_ref[...], c_ref[...],
                       preferred_element_type=jnp.float32) + c0_ref[...]


def _tc_mm(edge_attr, Cm, c0):
  return pl.pallas_call(
      _tc_mm_body,
      grid=(E // _MBLK,),
      in_specs=[
          pl.BlockSpec((_MBLK, D), lambda i: (i, 0)),
          pl.BlockSpec((D, D), lambda i: (0, 0)),
          pl.BlockSpec((1, D), lambda i: (0, 0)),
      ],
      out_specs=pl.BlockSpec((_MBLK, D), lambda i: (i, 0)),
      out_shape=jax.ShapeDtypeStruct((E, D), jnp.float32),
  )(edge_attr, Cm, c0)


def _tc_deg_body(p_ref, dinv_ref, dinv2_ref):
  s = p_ref[0, :] + p_ref[1, :] + 1.0
  dinv_ref[...] = lax.rsqrt(s)
  dinv2_ref[...] = 1.0 / s


def _tc_deg(degp):
  return pl.pallas_call(
      _tc_deg_body,
      out_shape=[
          jax.ShapeDtypeStruct((NP,), jnp.float32),
          jax.ShapeDtypeStruct((NP,), jnp.float32),
      ],
  )(degp)


@functools.partial(
    pl.kernel,
    out_type=jax.ShapeDtypeStruct((NC, NP), jnp.float32),
    mesh=_mesh,
    compiler_params=_sc_params,
    scratch_types=[
        pltpu.VMEM((K,), jnp.int32),
        pltpu.VMEM((K,), jnp.float32),
        pltpu.VMEM((RPS,), jnp.float32),
        pltpu.VMEM_SHARED((NP,), jnp.float32),
    ],
)
def _sc_indeg(dst_hbm, out_hbm, idx_v, ones_v, zb, deg_sh):
  cid = lax.axis_index("c")
  sid = lax.axis_index("s")
  wid = cid * NS + sid

  def zset(i):
    zb[pl.ds(i * 16, 16)] = jnp.zeros((16,), jnp.float32)
  _floop(RPS // 16, zset)

  def oset(i):
    ones_v[pl.ds(i * 16, 16)] = jnp.ones((16,), jnp.float32)
  _floop(K // 16, oset)

  pltpu.sync_copy(zb, deg_sh.at[pl.ds(sid * RPS, RPS)])
  plsc.subcore_barrier()

  def chunk(j):
    base = wid * EPW + j * K
    pltpu.sync_copy(dst_hbm.at[pl.ds(base, K)], idx_v)
    pltpu.sync_copy(ones_v, deg_sh.at[idx_v], add=True)
  _floop(NCH, chunk)

  plsc.subcore_barrier()
  pltpu.sync_copy(deg_sh.at[pl.ds(sid * RPS, RPS)],
                  out_hbm.at[cid, pl.ds(sid * RPS, RPS)])


@functools.partial(
    pl.kernel,
    out_type=jax.ShapeDtypeStruct((NP, D), jnp.float32),
    mesh=_mesh,
    compiler_params=_sc_params,
    scratch_types=[
        pltpu.VMEM((RPW,), jnp.float32),
        pltpu.VMEM((K, D), jnp.float32),
    ],
)
def _sc_scale1(u_hbm, dinv_hbm, g1_hbm, dv, ub):
  cid = lax.axis_index("c")
  sid = lax.axis_index("s")
  wid = cid * NS + sid
  r0 = wid * RPW
  pltpu.sync_copy(dinv_hbm.at[pl.ds(r0, RPW)], dv)
  for c in range(RPW // K):
    pltpu.sync_copy(u_hbm.at[pl.ds(r0 + c * K, K), :], ub)

    def row(i):
      db = _bcast16(dv, c * K + i)
      for f in range(D // 16):
        sl = pl.ds(f * 16, 16)
        ub[i, sl] = ub[i, sl] * db
    _floop(K, row)
    pltpu.sync_copy(ub, g1_hbm.at[pl.ds(r0 + c * K, K), :])


@functools.partial(
    pl.kernel,
    out_type=jax.ShapeDtypeStruct((NC, NP, D), jnp.float32),
    mesh=_mesh,
    compiler_params=_sc_params,
    scratch_types=[
        pltpu.VMEM((2, K), jnp.int32),
        pltpu.VMEM((2, K), jnp.int32),
        pltpu.VMEM((2, K), jnp.float32),
        pltpu.VMEM((K, D), jnp.float32),
        pltpu.VMEM((2, K, D), jnp.float32),
        pltpu.VMEM((K, D), jnp.float32),
        pltpu.VMEM_SHARED((NP, D), jnp.float32),
    ] + [pltpu.SemaphoreType.DMA] * 11,
)
def _sc_main(ea_hbm, src_hbm, dst_hbm, v_hbm, g1_hbm, dinv_hbm,
             tpart_hbm,
             src_v, dst_v, w_v, sb, vb, gb, tsh,
             si0, si1, se0, se1, sv0, sv1, sg, sw0, sw1, ss0, ss1):
  cid = lax.axis_index("c")
  sid = lax.axis_index("s")
  wid = cid * NS + sid
  sem_i = (si0, si1)
  sem_e = (se0, se1)
  sem_v = (sv0, sv1)
  sem_w = (sw0, sw1)
  sem_s = (ss0, ss1)

  def zrow(i):
    for f in range(D // 16):
      vb[0, i, pl.ds(f * 16, 16)] = jnp.zeros((16,), jnp.float32)
  _floop(K, zrow)
  for q in range(RPS // K):
    pltpu.sync_copy(vb.at[0], tsh.at[pl.ds(sid * RPS + q * K, K), :])
  plsc.subcore_barrier()

  def group(grp):
    g0 = grp * GC

    def stage(g):
      s = g % 2
      base = wid * EPW + (g0 + g) * K
      return (pltpu.async_copy(src_hbm.at[pl.ds(base, K)], src_v.at[s],
                               sem_i[s]),
              pltpu.async_copy(dst_hbm.at[pl.ds(base, K)], dst_v.at[s],
                               sem_i[s]))

    def ea_load(g):
      base = wid * EPW + (g0 + g) * K
      return pltpu.async_copy(ea_hbm.at[pl.ds(base, K), :], sb, se0)

    def gathers(g):
      s = g % 2
      return (pltpu.async_copy(v_hbm.at[dst_v.at[s]], vb.at[s], sem_v[s]),
              pltpu.async_copy(dinv_hbm.at[src_v.at[s]], w_v.at[s],
                               sem_w[s]))

    def g1_gather(g):
      s = g % 2
      return pltpu.async_copy(g1_hbm.at[src_v.at[s]], gb, sg)

    st = {0: stage(0), 1: stage(1)}
    ea = {0: ea_load(0)}
    st[0][0].wait()
    st[0][1].wait()
    ga = {0: gathers(0)}
    gg = {0: g1_gather(0)}
    sc = {}
    for g in range(GC):
      s = g % 2
      if g + 1 < GC:
        if g >= 1:
          sc[g - 1].wait()
        st[g + 1][0].wait()
        st[g + 1][1].wait()
        ga[g + 1] = gathers(g + 1)
      ea[g].wait()
      ga[g][0].wait()
      ga[g][1].wait()
      gg[g].wait()

      def row(i):
        wb = _bcast16(w_v.at[s], i)
        for f in range(D // 16):
          sl = pl.ds(f * 16, 16)
          vb[s, i, sl] = (sb[i, sl] + vb[s, i, sl]) * wb + gb[i, sl]
      _floop(K, row)

      sc[g] = pltpu.async_copy(vb.at[s], tsh.at[dst_v.at[s]], ss0 if s == 0
                               else ss1, add=True)
      if g + 1 < GC:
        ea[g + 1] = ea_load(g + 1)
        gg[g + 1] = g1_gather(g + 1)
      if g + 2 < GC:
        st[g + 2] = stage(g + 2)
    sc[GC - 2].wait()
    sc[GC - 1].wait()
  _floop(GRP, group)

  plsc.subcore_barrier()
  pltpu.sync_copy(tsh.at[pl.ds(sid * RPS, RPS), :],
                  tpart_hbm.at[cid, pl.ds(sid * RPS, RPS), :])


@functools.partial(
    pl.kernel,
    out_type=jax.ShapeDtypeStruct((NP, D), jnp.float32),
    mesh=_mesh,
    compiler_params=_sc_params,
    scratch_types=[
        pltpu.VMEM((RPW,), jnp.float32),
        pltpu.VMEM((RPW,), jnp.float32),
        pltpu.VMEM((K, D), jnp.float32),
        pltpu.VMEM((K, D), jnp.float32),
        pltpu.VMEM((K, D), jnp.float32),
    ],
)
def _sc_scale2(tpart_hbm, u_hbm, dinv_hbm, dinv2_hbm, g_hbm,
               dv, dv2, t0, t1, ub):
  cid = lax.axis_index("c")
  sid = lax.axis_index("s")
  wid = cid * NS + sid
  r0 = wid * RPW
  pltpu.sync_copy(dinv_hbm.at[pl.ds(r0, RPW)], dv)
  pltpu.sync_copy(dinv2_hbm.at[pl.ds(r0, RPW)], dv2)
  for c in range(RPW // K):
    rb = r0 + c * K
    pltpu.sync_copy(tpart_hbm.at[0, pl.ds(rb, K), :], t0)
    pltpu.sync_copy(tpart_hbm.at[1, pl.ds(rb, K), :], t1)
    pltpu.sync_copy(u_hbm.at[pl.ds(rb, K), :], ub)

    def row(i):
      db = _bcast16(dv, c * K + i)
      d2b = _bcast16(dv2, c * K + i)
      for f in range(D // 16):
        sl = pl.ds(f * 16, 16)
        t0[i, sl] = (t0[i, sl] + t1[i, sl]) * db + ub[i, sl] * d2b
    _floop(K, row)
    pltpu.sync_copy(t0, g_hbm.at[pl.ds(rb, K), :])


@functools.partial(
    pl.kernel,
    out_type=jax.ShapeDtypeStruct((E, D), jnp.float32),
    mesh=_mesh,
    compiler_params=_sc_params,
    scratch_types=[
        pltpu.VMEM((2, K), jnp.int32),
        pltpu.VMEM((2, K), jnp.int32),
        pltpu.VMEM((2, K), jnp.float32),
        pltpu.VMEM((D,), jnp.float32),
        pltpu.VMEM((2, K, D), jnp.float32),
        pltpu.VMEM((2, K, D), jnp.float32),
        pltpu.VMEM((2, K, D), jnp.float32),
        pltpu.VMEM((2, K, D), jnp.float32),
    ] + [pltpu.SemaphoreType.DMA] * 12,
)
def _sc_out(ea_hbm, src_hbm, dst_hbm, v_hbm, g_hbm, dinv2_hbm, b2_hbm,
            out_hbm, src_v, dst_v, w2_v, b2v, sb, vb, gb, ob,
            si0, si1, se0, se1, sv0, sv1, sg0, sg1, sw0, sw1, so0, so1):
  cid = lax.axis_index("c")
  sid = lax.axis_index("s")
  wid = cid * NS + sid
  sem_i = (si0, si1)
  sem_e = (se0, se1)
  sem_v = (sv0, sv1)
  sem_g = (sg0, sg1)
  sem_w = (sw0, sw1)
  sem_o = (so0, so1)
  pltpu.sync_copy(b2_hbm, b2v)

  def group(grp):
    g0 = grp * GC

    def stage(g):
      s = g % 2
      base = wid * EPW + (g0 + g) * K
      return (pltpu.async_copy(src_hbm.at[pl.ds(base, K)], src_v.at[s],
                               sem_i[s]),
              pltpu.async_copy(dst_hbm.at[pl.ds(base, K)], dst_v.at[s],
                               sem_i[s]),
              pltpu.async_copy(ea_hbm.at[pl.ds(base, K), :], sb.at[s],
                               sem_e[s]))

    def gathers(g):
      s = g % 2
      return (pltpu.async_copy(v_hbm.at[dst_v.at[s]], vb.at[s], sem_v[s]),
              pltpu.async_copy(g_hbm.at[src_v.at[s]], gb.at[s], sem_g[s]),
              pltpu.async_copy(dinv2_hbm.at[src_v.at[s]], w2_v.at[s],
                               sem_w[s]))

    st = {0: stage(0), 1: stage(1)}
    st[0][0].wait()
    st[0][1].wait()
    ga = {0: gathers(0)}
    wr = {}
    for g in range(GC):
      s = g % 2
      if g >= 2:
        wr[g - 2].wait()
      if g + 1 < GC:
        st[g + 1][0].wait()
        st[g + 1][1].wait()
        ga[g + 1] = gathers(g + 1)
      st[g][2].wait()
      ga[g][0].wait()
      ga[g][1].wait()
      ga[g][2].wait()

      def row(i):
        w2b = _bcast16(w2_v.at[s], i)
        for f in range(D // 16):
          sl = pl.ds(f * 16, 16)
          o = (sb[s, i, sl] + vb[s, i, sl]) * w2b + gb[s, i, sl] + b2v[sl]
          ob[s, i, sl] = jnp.maximum(o, 0.01 * o)
      _floop(K, row)

      base = wid * EPW + (g0 + g) * K
      wr[g] = pltpu.async_copy(ob.at[s], out_hbm.at[pl.ds(base, K), :],
                               sem_o[s])
      if g + 2 < GC:
        st[g + 2] = stage(g + 2)
    wr[GC - 2].wait()
    wr[GC - 1].wait()
  _floop(GRP, group)


@jax.jit
def kernel(x, edge_index, batch, edge_attr, W1, b1, W2, b2):
  src = edge_index[0]
  dst = edge_index[1]
  xp = jnp.pad(x, ((0, NP - N), (0, 0)))
  u, v, Cm, c0 = _tc_prep(xp, W1, W2, b1.reshape(1, D))
  ea = _tc_mm(edge_attr, Cm, c0)
  degp = _sc_indeg(dst)
  dinv, dinv2 = _tc_deg(degp)
  g1 = _sc_scale1(u, dinv)
  tpart = _sc_main(ea, src, dst, v, g1, dinv)
  G = _sc_scale2(tpart, u, dinv, dinv2)
  return _sc_out(ea, src, dst, v, G, dinv2, b2)


# speedup vs baseline: 3.0467x; 1.0418x over previous
import functools

import jax
import jax.numpy as jnp
from jax import lax
from jax.experimental import pallas as pl
from jax.experimental.pallas import tpu as pltpu
from jax.experimental.pallas import tpu_sc as plsc

N = 10000
E = 320000
D = 128
NP = 10240
NC = 2
NS = 16
NW = NC * NS
EPW = E // NW
K = 80
NCH = EPW // K
GRP = 5
GC = NCH // GRP
RPW = NP // NW
RPS = NP // NS

_mesh = plsc.VectorSubcoreMesh(core_axis_name="c", subcore_axis_name="s")
_sc_params = pltpu.CompilerParams(needs_layout_passes=False)


def _floop(n, body):
  lax.fori_loop(0, n, lambda i, c: (body(i), c)[1], 0)


def _bcast16(val_ref, i):
  return plsc.load_gather(val_ref, [jnp.full((16,), i, jnp.int32)])


def _tc_prep_body(x_ref, w1_ref, w2_ref, b1_ref, u_ref, v_ref, c_ref, c0_ref):
  W2 = w2_ref[...]
  xv = x_ref[...]
  A = jnp.dot(w1_ref[0:D, :], W2, preferred_element_type=jnp.float32)
  B = jnp.dot(w1_ref[D:2 * D, :], W2, preferred_element_type=jnp.float32)
  u_ref[...] = jnp.dot(xv, A, preferred_element_type=jnp.float32)
  v_ref[...] = jnp.dot(xv, B, preferred_element_type=jnp.float32)
  c_ref[...] = jnp.dot(w1_ref[2 * D:3 * D, :], W2,
                       preferred_element_type=jnp.float32)
  c0_ref[...] = jnp.dot(b1_ref[...], W2, preferred_element_type=jnp.float32)


def _tc_prep(xp, W1, W2, b1row):
  return pl.pallas_call(
      _tc_prep_body,
      out_shape=[
          jax.ShapeDtypeStruct((NP, D), jnp.float32),
          jax.ShapeDtypeStruct((NP, D), jnp.float32),
          jax.ShapeDtypeStruct((D, D), jnp.float32),
          jax.ShapeDtypeStruct((1, D), jnp.float32),
      ],
  )(xp, W1, W2, b1row)


_MBLK = 2000


def _tc_mm_body(ea_ref, c_ref, c0_ref, o_ref):
  o_ref[...] = jnp.dot(ea_ref[...], c_ref[...],
                       preferred_element_type=jnp.float32) + c0_ref[...]


def _tc_mm(edge_attr, Cm, c0):
  return pl.pallas_call(
      _tc_mm_body,
      grid=(E // _MBLK,),
      in_specs=[
          pl.BlockSpec((_MBLK, D), lambda i: (i, 0)),
          pl.BlockSpec((D, D), lambda i: (0, 0)),
          pl.BlockSpec((1, D), lambda i: (0, 0)),
      ],
      out_specs=pl.BlockSpec((_MBLK, D), lambda i: (i, 0)),
      out_shape=jax.ShapeDtypeStruct((E, D), jnp.float32),
  )(edge_attr, Cm, c0)


def _tc_deg_body(p_ref, dinv_ref, dinv2_ref):
  s = p_ref[0, :] + p_ref[1, :] + 1.0
  dinv_ref[...] = lax.rsqrt(s)
  dinv2_ref[...] = 1.0 / s


def _tc_deg(degp):
  return pl.pallas_call(
      _tc_deg_body,
      out_shape=[
          jax.ShapeDtypeStruct((NP,), jnp.float32),
          jax.ShapeDtypeStruct((NP,), jnp.float32),
      ],
  )(degp)


@functools.partial(
    pl.kernel,
    out_type=jax.ShapeDtypeStruct((NC, NP), jnp.float32),
    mesh=_mesh,
    compiler_params=_sc_params,
    scratch_types=[
        pltpu.VMEM((K,), jnp.int32),
        pltpu.VMEM((K,), jnp.float32),
        pltpu.VMEM((RPS,), jnp.float32),
        pltpu.VMEM_SHARED((NP,), jnp.float32),
    ],
)
def _sc_indeg(dst_hbm, out_hbm, idx_v, ones_v, zb, deg_sh):
  cid = lax.axis_index("c")
  sid = lax.axis_index("s")
  wid = cid * NS + sid

  def zset(i):
    zb[pl.ds(i * 16, 16)] = jnp.zeros((16,), jnp.float32)
  _floop(RPS // 16, zset)

  def oset(i):
    ones_v[pl.ds(i * 16, 16)] = jnp.ones((16,), jnp.float32)
  _floop(K // 16, oset)

  pltpu.sync_copy(zb, deg_sh.at[pl.ds(sid * RPS, RPS)])
  plsc.subcore_barrier()

  def chunk(j):
    base = wid * EPW + j * K
    pltpu.sync_copy(dst_hbm.at[pl.ds(base, K)], idx_v)
    pltpu.sync_copy(ones_v, deg_sh.at[idx_v], add=True)
  _floop(NCH, chunk)

  plsc.subcore_barrier()
  pltpu.sync_copy(deg_sh.at[pl.ds(sid * RPS, RPS)],
                  out_hbm.at[cid, pl.ds(sid * RPS, RPS)])


@functools.partial(
    pl.kernel,
    out_type=jax.ShapeDtypeStruct((NP, D), jnp.float32),
    mesh=_mesh,
    compiler_params=_sc_params,
    scratch_types=[
        pltpu.VMEM((RPW,), jnp.float32),
        pltpu.VMEM((K, D), jnp.float32),
    ],
)
def _sc_scale1(u_hbm, dinv_hbm, g1_hbm, dv, ub):
  cid = lax.axis_index("c")
  sid = lax.axis_index("s")
  wid = cid * NS + sid
  r0 = wid * RPW
  pltpu.sync_copy(dinv_hbm.at[pl.ds(r0, RPW)], dv)
  for c in range(RPW // K):
    pltpu.sync_copy(u_hbm.at[pl.ds(r0 + c * K, K), :], ub)

    def row(i):
      db = _bcast16(dv, c * K + i)
      for f in range(D // 16):
        sl = pl.ds(f * 16, 16)
        ub[i, sl] = ub[i, sl] * db
    _floop(K, row)
    pltpu.sync_copy(ub, g1_hbm.at[pl.ds(r0 + c * K, K), :])


@functools.partial(
    pl.kernel,
    out_type=jax.ShapeDtypeStruct((NC, NP, D), jnp.float32),
    mesh=_mesh,
    compiler_params=_sc_params,
    scratch_types=[
        pltpu.VMEM((2, K), jnp.int32),
        pltpu.VMEM((2, K), jnp.int32),
        pltpu.VMEM((2, K), jnp.float32),
        pltpu.VMEM((K, D), jnp.float32),
        pltpu.VMEM((2, K, D), jnp.float32),
        pltpu.VMEM((K, D), jnp.float32),
        pltpu.VMEM_SHARED((NP, D), jnp.float32),
    ] + [pltpu.SemaphoreType.DMA] * 11,
)
def _sc_main(ea_hbm, src_hbm, dst_hbm, v_hbm, g1_hbm, dinv_hbm,
             tpart_hbm,
             src_v, dst_v, w_v, sb, vb, gb, tsh,
             si0, si1, se0, se1, sv0, sv1, sg, sw0, sw1, ss0, ss1):
  cid = lax.axis_index("c")
  sid = lax.axis_index("s")
  wid = cid * NS + sid
  sem_i = (si0, si1)
  sem_e = (se0, se1)
  sem_v = (sv0, sv1)
  sem_w = (sw0, sw1)
  sem_s = (ss0, ss1)

  def zrow(i):
    for f in range(D // 16):
      vb[0, i, pl.ds(f * 16, 16)] = jnp.zeros((16,), jnp.float32)
  _floop(K, zrow)
  for q in range(RPS // K):
    pltpu.sync_copy(vb.at[0], tsh.at[pl.ds(sid * RPS + q * K, K), :])
  plsc.subcore_barrier()

  def group(grp):
    g0 = grp * GC

    def stage(g):
      s = g % 2
      base = wid * EPW + (g0 + g) * K
      return (pltpu.async_copy(src_hbm.at[pl.ds(base, K)], src_v.at[s],
                               sem_i[s]),
              pltpu.async_copy(dst_hbm.at[pl.ds(base, K)], dst_v.at[s],
                               sem_i[s]))

    def ea_load(g):
      base = wid * EPW + (g0 + g) * K
      return pltpu.async_copy(ea_hbm.at[pl.ds(base, K), :], sb, se0)

    def gathers(g):
      s = g % 2
      return (pltpu.async_copy(v_hbm.at[dst_v.at[s]], vb.at[s], sem_v[s]),
              pltpu.async_copy(dinv_hbm.at[src_v.at[s]], w_v.at[s],
                               sem_w[s]))

    def g1_gather(g):
      s = g % 2
      return pltpu.async_copy(g1_hbm.at[src_v.at[s]], gb, sg)

    st = {0: stage(0), 1: stage(1)}
    ea = {0: ea_load(0)}
    st[0][0].wait()
    st[0][1].wait()
    ga = {0: gathers(0)}
    gg = {0: g1_gather(0)}
    sc = {}
    for g in range(GC):
      s = g % 2
      if g + 1 < GC:
        if g >= 1:
          sc[g - 1].wait()
        st[g + 1][0].wait()
        st[g + 1][1].wait()
        ga[g + 1] = gathers(g + 1)
      ea[g].wait()
      ga[g][0].wait()
      ga[g][1].wait()
      gg[g].wait()

      def row(i):
        wb = _bcast16(w_v.at[s], i)
        for f in range(D // 16):
          sl = pl.ds(f * 16, 16)
          vb[s, i, sl] = (sb[i, sl] + vb[s, i, sl]) * wb + gb[i, sl]
      _floop(K, row)

      sc[g] = pltpu.async_copy(vb.at[s], tsh.at[dst_v.at[s]], ss0 if s == 0
                               else ss1, add=True)
      if g + 1 < GC:
        ea[g + 1] = ea_load(g + 1)
        gg[g + 1] = g1_gather(g + 1)
      if g + 2 < GC:
        st[g + 2] = stage(g + 2)
    sc[GC - 2].wait()
    sc[GC - 1].wait()
  _floop(GRP, group)

  plsc.subcore_barrier()
  pltpu.sync_copy(tsh.at[pl.ds(sid * RPS, RPS), :],
                  tpart_hbm.at[cid, pl.ds(sid * RPS, RPS), :])


@functools.partial(
    pl.kernel,
    out_type=jax.ShapeDtypeStruct((NP, D), jnp.float32),
    mesh=_mesh,
    compiler_params=_sc_params,
    scratch_types=[
        pltpu.VMEM((RPW,), jnp.float32),
        pltpu.VMEM((RPW,), jnp.float32),
        pltpu.VMEM((D,), jnp.float32),
        pltpu.VMEM((K, D), jnp.float32),
        pltpu.VMEM((K, D), jnp.float32),
        pltpu.VMEM((K, D), jnp.float32),
    ],
)
def _sc_scale2(tpart_hbm, u_hbm, dinv_hbm, dinv2_hbm, b2_hbm, g_hbm,
               dv, dv2, b2v, t0, t1, ub):
  cid = lax.axis_index("c")
  sid = lax.axis_index("s")
  wid = cid * NS + sid
  r0 = wid * RPW
  pltpu.sync_copy(dinv_hbm.at[pl.ds(r0, RPW)], dv)
  pltpu.sync_copy(dinv2_hbm.at[pl.ds(r0, RPW)], dv2)
  pltpu.sync_copy(b2_hbm, b2v)
  for c in range(RPW // K):
    rb = r0 + c * K
    pltpu.sync_copy(tpart_hbm.at[0, pl.ds(rb, K), :], t0)
    pltpu.sync_copy(tpart_hbm.at[1, pl.ds(rb, K), :], t1)
    pltpu.sync_copy(u_hbm.at[pl.ds(rb, K), :], ub)

    def row(i):
      db = _bcast16(dv, c * K + i)
      d2b = _bcast16(dv2, c * K + i)
      for f in range(D // 16):
        sl = pl.ds(f * 16, 16)
        t0[i, sl] = (t0[i, sl] + t1[i, sl]) * db + ub[i, sl] * d2b + b2v[sl]
    _floop(K, row)
    pltpu.sync_copy(t0, g_hbm.at[pl.ds(rb, K), :])


@functools.partial(
    pl.kernel,
    out_type=jax.ShapeDtypeStruct((E, D), jnp.float32),
    mesh=_mesh,
    compiler_params=_sc_params,
    scratch_types=[
        pltpu.VMEM((2, K), jnp.int32),
        pltpu.VMEM((2, K), jnp.int32),
        pltpu.VMEM((2, K), jnp.float32),
        pltpu.VMEM((2, K, D), jnp.float32),
        pltpu.VMEM((2, K, D), jnp.float32),
        pltpu.VMEM((2, K, D), jnp.float32),
        pltpu.VMEM((2, K, D), jnp.float32),
    ] + [pltpu.SemaphoreType.DMA] * 12,
)
def _sc_out(ea_hbm, src_hbm, dst_hbm, v_hbm, g_hbm, dinv2_hbm,
            out_hbm, src_v, dst_v, w2_v, sb, vb, gb, ob,
            si0, si1, se0, se1, sv0, sv1, sg0, sg1, sw0, sw1, so0, so1):
  cid = lax.axis_index("c")
  sid = lax.axis_index("s")
  wid = cid * NS + sid
  sem_i = (si0, si1)
  sem_e = (se0, se1)
  sem_v = (sv0, sv1)
  sem_g = (sg0, sg1)
  sem_w = (sw0, sw1)
  sem_o = (so0, so1)

  def group(grp):
    g0 = grp * GC

    def stage(g):
      s = g % 2
      base = wid * EPW + (g0 + g) * K
      return (pltpu.async_copy(src_hbm.at[pl.ds(base, K)], src_v.at[s],
                               sem_i[s]),
              pltpu.async_copy(dst_hbm.at[pl.ds(base, K)], dst_v.at[s],
                               sem_i[s]),
              pltpu.async_copy(ea_hbm.at[pl.ds(base, K), :], sb.at[s],
                               sem_e[s]))

    def gathers(g):
      s = g % 2
      return (pltpu.async_copy(v_hbm.at[dst_v.at[s]], vb.at[s], sem_v[s]),
              pltpu.async_copy(g_hbm.at[src_v.at[s]], gb.at[s], sem_g[s]),
              pltpu.async_copy(dinv2_hbm.at[src_v.at[s]], w2_v.at[s],
                               sem_w[s]))

    st = {0: stage(0), 1: stage(1)}
    st[0][0].wait()
    st[0][1].wait()
    ga = {0: gathers(0)}
    wr = {}
    for g in range(GC):
      s = g % 2
      if g >= 2:
        wr[g - 2].wait()
      if g + 1 < GC:
        st[g + 1][0].wait()
        st[g + 1][1].wait()
        ga[g + 1] = gathers(g + 1)
      st[g][2].wait()
      ga[g][0].wait()
      ga[g][1].wait()
      ga[g][2].wait()

      def row(i):
        w2b = _bcast16(w2_v.at[s], i)
        for f in range(D // 16):
          sl = pl.ds(f * 16, 16)
          o = (sb[s, i, sl] + vb[s, i, sl]) * w2b + gb[s, i, sl]
          ob[s, i, sl] = jnp.maximum(o, 0.01 * o)
      _floop(K, row)

      base = wid * EPW + (g0 + g) * K
      wr[g] = pltpu.async_copy(ob.at[s], out_hbm.at[pl.ds(base, K), :],
                               sem_o[s])
      if g + 2 < GC:
        st[g + 2] = stage(g + 2)
    wr[GC - 2].wait()
    wr[GC - 1].wait()
  _floop(GRP, group)


@jax.jit
def kernel(x, edge_index, batch, edge_attr, W1, b1, W2, b2):
  src = edge_index[0]
  dst = edge_index[1]
  xp = jnp.pad(x, ((0, NP - N), (0, 0)))
  u, v, Cm, c0 = _tc_prep(xp, W1, W2, b1.reshape(1, D))
  ea = _tc_mm(edge_attr, Cm, c0)
  degp = _sc_indeg(dst)
  dinv, dinv2 = _tc_deg(degp)
  g1 = _sc_scale1(u, dinv)
  tpart = _sc_main(ea, src, dst, v, g1, dinv)
  G = _sc_scale2(tpart, u, dinv, dinv2, b2)
  return _sc_out(ea, src, dst, v, G, dinv2)


# speedup vs baseline: 6.1434x; 2.0164x over previous
import functools

import jax
import jax.numpy as jnp
from jax import lax
from jax.experimental import pallas as pl
from jax.experimental.pallas import tpu as pltpu
from jax.experimental.pallas import tpu_sc as plsc

N = 10000
E = 320000
D = 128
NP = 10240
NC = 2
NS = 16
NW = NC * NS
EPW = E // NW
K = 80
NCH = EPW // K
GRP = 5
GC = NCH // GRP
RPW = NP // NW
RPS = NP // NS

_mesh = plsc.VectorSubcoreMesh(core_axis_name="c", subcore_axis_name="s")
_sc_params = pltpu.CompilerParams(needs_layout_passes=False)


def _floop(n, body):
  lax.fori_loop(0, n, lambda i, c: (body(i), c)[1], 0)


def _bcast16(val_ref, i):
  return plsc.load_gather(val_ref, [jnp.full((16,), i, jnp.int32)])


def _tc_prep_body(x_ref, w1_ref, w2_ref, b1_ref, u_ref, v_ref, c_ref, c0_ref):
  W2 = w2_ref[...]
  xv = x_ref[...]
  A = jnp.dot(w1_ref[0:D, :], W2, preferred_element_type=jnp.float32)
  B = jnp.dot(w1_ref[D:2 * D, :], W2, preferred_element_type=jnp.float32)
  u_ref[...] = jnp.dot(xv, A, preferred_element_type=jnp.float32)
  v_ref[...] = jnp.dot(xv, B, preferred_element_type=jnp.float32)
  c_ref[...] = jnp.dot(w1_ref[2 * D:3 * D, :], W2,
                       preferred_element_type=jnp.float32)
  c0_ref[...] = jnp.dot(b1_ref[...], W2, preferred_element_type=jnp.float32)


def _tc_prep(xp, W1, W2, b1row):
  return pl.pallas_call(
      _tc_prep_body,
      out_shape=[
          jax.ShapeDtypeStruct((NP, D), jnp.float32),
          jax.ShapeDtypeStruct((NP, D), jnp.float32),
          jax.ShapeDtypeStruct((D, D), jnp.float32),
          jax.ShapeDtypeStruct((1, D), jnp.float32),
      ],
  )(xp, W1, W2, b1row)


_MBLK = 2000


def _tc_mm_body(ea_ref, c_ref, c0_ref, o_ref):
  o_ref[...] = jnp.dot(ea_ref[...], c_ref[...],
                       preferred_element_type=jnp.float32) + c0_ref[...]


def _tc_mm(edge_attr, Cm, c0):
  return pl.pallas_call(
      _tc_mm_body,
      grid=(E // _MBLK,),
      in_specs=[
          pl.BlockSpec((_MBLK, D), lambda i: (i, 0)),
          pl.BlockSpec((D, D), lambda i: (0, 0)),
          pl.BlockSpec((1, D), lambda i: (0, 0)),
      ],
      out_specs=pl.BlockSpec((_MBLK, D), lambda i: (i, 0)),
      out_shape=jax.ShapeDtypeStruct((E, D), jnp.float32),
  )(edge_attr, Cm, c0)


def _tc_deg_body(p_ref, dinv_ref, dinv2_ref):
  s = p_ref[0, :] + p_ref[1, :] + 1.0
  dinv_ref[...] = lax.rsqrt(s)
  dinv2_ref[...] = 1.0 / s


def _tc_deg(degp):
  return pl.pallas_call(
      _tc_deg_body,
      out_shape=[
          jax.ShapeDtypeStruct((NP,), jnp.float32),
          jax.ShapeDtypeStruct((NP,), jnp.float32),
      ],
  )(degp)


@functools.partial(
    pl.kernel,
    out_type=jax.ShapeDtypeStruct((NC, NP), jnp.float32),
    mesh=_mesh,
    compiler_params=_sc_params,
    scratch_types=[
        pltpu.VMEM((K,), jnp.int32),
        pltpu.VMEM((K,), jnp.float32),
        pltpu.VMEM((RPS,), jnp.float32),
        pltpu.VMEM_SHARED((NP,), jnp.float32),
    ],
)
def _sc_indeg(dst_hbm, out_hbm, idx_v, ones_v, zb, deg_sh):
  cid = lax.axis_index("c")
  sid = lax.axis_index("s")
  wid = cid * NS + sid

  def zset(i):
    zb[pl.ds(i * 16, 16)] = jnp.zeros((16,), jnp.float32)
  _floop(RPS // 16, zset)

  def oset(i):
    ones_v[pl.ds(i * 16, 16)] = jnp.ones((16,), jnp.float32)
  _floop(K // 16, oset)

  pltpu.sync_copy(zb, deg_sh.at[pl.ds(sid * RPS, RPS)])
  plsc.subcore_barrier()

  def chunk(j):
    base = wid * EPW + j * K
    pltpu.sync_copy(dst_hbm.at[pl.ds(base, K)], idx_v)
    pltpu.sync_copy(ones_v, deg_sh.at[idx_v], add=True)
  _floop(NCH, chunk)

  plsc.subcore_barrier()
  pltpu.sync_copy(deg_sh.at[pl.ds(sid * RPS, RPS)],
                  out_hbm.at[cid, pl.ds(sid * RPS, RPS)])


@functools.partial(
    pl.kernel,
    out_type=jax.ShapeDtypeStruct((NP, D), jnp.float32),
    mesh=_mesh,
    compiler_params=_sc_params,
    scratch_types=[
        pltpu.VMEM((RPW,), jnp.float32),
        pltpu.VMEM((K, D), jnp.float32),
    ],
)
def _sc_scale1(u_hbm, dinv_hbm, g1_hbm, dv, ub):
  cid = lax.axis_index("c")
  sid = lax.axis_index("s")
  wid = cid * NS + sid
  r0 = wid * RPW
  pltpu.sync_copy(dinv_hbm.at[pl.ds(r0, RPW)], dv)
  for c in range(RPW // K):
    pltpu.sync_copy(u_hbm.at[pl.ds(r0 + c * K, K), :], ub)

    def row(i):
      db = _bcast16(dv, c * K + i)
      for f in range(D // 16):
        sl = pl.ds(f * 16, 16)
        ub[i, sl] = ub[i, sl] * db
    plsc.parallel_loop(0, K)(row)
    pltpu.sync_copy(ub, g1_hbm.at[pl.ds(r0 + c * K, K), :])


@functools.partial(
    pl.kernel,
    out_type=jax.ShapeDtypeStruct((NC, NP, D), jnp.float32),
    mesh=_mesh,
    compiler_params=_sc_params,
    scratch_types=[
        pltpu.VMEM((2, K), jnp.int32),
        pltpu.VMEM((2, K), jnp.int32),
        pltpu.VMEM((2, K), jnp.float32),
        pltpu.VMEM((K, D), jnp.float32),
        pltpu.VMEM((2, K, D), jnp.float32),
        pltpu.VMEM((K, D), jnp.float32),
        pltpu.VMEM_SHARED((NP, D), jnp.float32),
    ] + [pltpu.SemaphoreType.DMA] * 11,
)
def _sc_main(ea_hbm, src_hbm, dst_hbm, v_hbm, g1_hbm, dinv_hbm,
             tpart_hbm,
             src_v, dst_v, w_v, sb, vb, gb, tsh,
             si0, si1, se0, se1, sv0, sv1, sg, sw0, sw1, ss0, ss1):
  cid = lax.axis_index("c")
  sid = lax.axis_index("s")
  wid = cid * NS + sid
  sem_i = (si0, si1)
  sem_e = (se0, se1)
  sem_v = (sv0, sv1)
  sem_w = (sw0, sw1)
  sem_s = (ss0, ss1)

  def zrow(i):
    for f in range(D // 16):
      vb[0, i, pl.ds(f * 16, 16)] = jnp.zeros((16,), jnp.float32)
  _floop(K, zrow)
  for q in range(RPS // K):
    pltpu.sync_copy(vb.at[0], tsh.at[pl.ds(sid * RPS + q * K, K), :])
  plsc.subcore_barrier()

  def group(grp):
    g0 = grp * GC

    def stage(g):
      s = g % 2
      base = wid * EPW + (g0 + g) * K
      return (pltpu.async_copy(src_hbm.at[pl.ds(base, K)], src_v.at[s],
                               sem_i[s]),
              pltpu.async_copy(dst_hbm.at[pl.ds(base, K)], dst_v.at[s],
                               sem_i[s]))

    def ea_load(g):
      base = wid * EPW + (g0 + g) * K
      return pltpu.async_copy(ea_hbm.at[pl.ds(base, K), :], sb, se0)

    def gathers(g):
      s = g % 2
      return (pltpu.async_copy(v_hbm.at[dst_v.at[s]], vb.at[s], sem_v[s]),
              pltpu.async_copy(dinv_hbm.at[src_v.at[s]], w_v.at[s],
                               sem_w[s]))

    def g1_gather(g):
      s = g % 2
      return pltpu.async_copy(g1_hbm.at[src_v.at[s]], gb, sg)

    st = {0: stage(0), 1: stage(1)}
    ea = {0: ea_load(0)}
    st[0][0].wait()
    st[0][1].wait()
    ga = {0: gathers(0)}
    gg = {0: g1_gather(0)}
    sc = {}
    for g in range(GC):
      s = g % 2
      if g + 1 < GC:
        if g >= 1:
          sc[g - 1].wait()
        st[g + 1][0].wait()
        st[g + 1][1].wait()
        ga[g + 1] = gathers(g + 1)
      ea[g].wait()
      ga[g][0].wait()
      ga[g][1].wait()
      gg[g].wait()

      def row(i):
        wb = _bcast16(w_v.at[s], i)
        for f in range(D // 16):
          sl = pl.ds(f * 16, 16)
          vb[s, i, sl] = (sb[i, sl] + vb[s, i, sl]) * wb + gb[i, sl]
      plsc.parallel_loop(0, K)(row)

      sc[g] = pltpu.async_copy(vb.at[s], tsh.at[dst_v.at[s]], ss0 if s == 0
                               else ss1, add=True)
      if g + 1 < GC:
        ea[g + 1] = ea_load(g + 1)
        gg[g + 1] = g1_gather(g + 1)
      if g + 2 < GC:
        st[g + 2] = stage(g + 2)
    sc[GC - 2].wait()
    sc[GC - 1].wait()
  _floop(GRP, group)

  plsc.subcore_barrier()
  pltpu.sync_copy(tsh.at[pl.ds(sid * RPS, RPS), :],
                  tpart_hbm.at[cid, pl.ds(sid * RPS, RPS), :])


@functools.partial(
    pl.kernel,
    out_type=jax.ShapeDtypeStruct((NP, D), jnp.float32),
    mesh=_mesh,
    compiler_params=_sc_params,
    scratch_types=[
        pltpu.VMEM((RPW,), jnp.float32),
        pltpu.VMEM((RPW,), jnp.float32),
        pltpu.VMEM((D,), jnp.float32),
        pltpu.VMEM((K, D), jnp.float32),
        pltpu.VMEM((K, D), jnp.float32),
        pltpu.VMEM((K, D), jnp.float32),
    ],
)
def _sc_scale2(tpart_hbm, u_hbm, dinv_hbm, dinv2_hbm, b2_hbm, g_hbm,
               dv, dv2, b2v, t0, t1, ub):
  cid = lax.axis_index("c")
  sid = lax.axis_index("s")
  wid = cid * NS + sid
  r0 = wid * RPW
  pltpu.sync_copy(dinv_hbm.at[pl.ds(r0, RPW)], dv)
  pltpu.sync_copy(dinv2_hbm.at[pl.ds(r0, RPW)], dv2)
  pltpu.sync_copy(b2_hbm, b2v)
  for c in range(RPW // K):
    rb = r0 + c * K
    pltpu.sync_copy(tpart_hbm.at[0, pl.ds(rb, K), :], t0)
    pltpu.sync_copy(tpart_hbm.at[1, pl.ds(rb, K), :], t1)
    pltpu.sync_copy(u_hbm.at[pl.ds(rb, K), :], ub)

    def row(i):
      db = _bcast16(dv, c * K + i)
      d2b = _bcast16(dv2, c * K + i)
      for f in range(D // 16):
        sl = pl.ds(f * 16, 16)
        t0[i, sl] = (t0[i, sl] + t1[i, sl]) * db + ub[i, sl] * d2b + b2v[sl]
    plsc.parallel_loop(0, K)(row)
    pltpu.sync_copy(t0, g_hbm.at[pl.ds(rb, K), :])


@functools.partial(
    pl.kernel,
    out_type=jax.ShapeDtypeStruct((E, D), jnp.float32),
    mesh=_mesh,
    compiler_params=_sc_params,
    scratch_types=[
        pltpu.VMEM((2, K), jnp.int32),
        pltpu.VMEM((2, K), jnp.int32),
        pltpu.VMEM((2, K), jnp.float32),
        pltpu.VMEM((2, K, D), jnp.float32),
        pltpu.VMEM((2, K, D), jnp.float32),
        pltpu.VMEM((2, K, D), jnp.float32),
        pltpu.VMEM((2, K, D), jnp.float32),
    ] + [pltpu.SemaphoreType.DMA] * 12,
)
def _sc_out(ea_hbm, src_hbm, dst_hbm, v_hbm, g_hbm, dinv2_hbm,
            out_hbm, src_v, dst_v, w2_v, sb, vb, gb, ob,
            si0, si1, se0, se1, sv0, sv1, sg0, sg1, sw0, sw1, so0, so1):
  cid = lax.axis_index("c")
  sid = lax.axis_index("s")
  wid = cid * NS + sid
  sem_i = (si0, si1)
  sem_e = (se0, se1)
  sem_v = (sv0, sv1)
  sem_g = (sg0, sg1)
  sem_w = (sw0, sw1)
  sem_o = (so0, so1)

  def group(grp):
    g0 = grp * GC

    def stage(g):
      s = g % 2
      base = wid * EPW + (g0 + g) * K
      return (pltpu.async_copy(src_hbm.at[pl.ds(base, K)], src_v.at[s],
                               sem_i[s]),
              pltpu.async_copy(dst_hbm.at[pl.ds(base, K)], dst_v.at[s],
                               sem_i[s]),
              pltpu.async_copy(ea_hbm.at[pl.ds(base, K), :], sb.at[s],
                               sem_e[s]))

    def gathers(g):
      s = g % 2
      return (pltpu.async_copy(v_hbm.at[dst_v.at[s]], vb.at[s], sem_v[s]),
              pltpu.async_copy(g_hbm.at[src_v.at[s]], gb.at[s], sem_g[s]),
              pltpu.async_copy(dinv2_hbm.at[src_v.at[s]], w2_v.at[s],
                               sem_w[s]))

    st = {0: stage(0), 1: stage(1)}
    st[0][0].wait()
    st[0][1].wait()
    ga = {0: gathers(0)}
    wr = {}
    for g in range(GC):
      s = g % 2
      if g >= 2:
        wr[g - 2].wait()
      if g + 1 < GC:
        st[g + 1][0].wait()
        st[g + 1][1].wait()
        ga[g + 1] = gathers(g + 1)
      st[g][2].wait()
      ga[g][0].wait()
      ga[g][1].wait()
      ga[g][2].wait()

      def row(i):
        w2b = _bcast16(w2_v.at[s], i)
        for f in range(D // 16):
          sl = pl.ds(f * 16, 16)
          o = (sb[s, i, sl] + vb[s, i, sl]) * w2b + gb[s, i, sl]
          ob[s, i, sl] = jnp.maximum(o, 0.01 * o)
      plsc.parallel_loop(0, K)(row)

      base = wid * EPW + (g0 + g) * K
      wr[g] = pltpu.async_copy(ob.at[s], out_hbm.at[pl.ds(base, K), :],
                               sem_o[s])
      if g + 2 < GC:
        st[g + 2] = stage(g + 2)
    wr[GC - 2].wait()
    wr[GC - 1].wait()
  _floop(GRP, group)


@jax.jit
def kernel(x, edge_index, batch, edge_attr, W1, b1, W2, b2):
  src = edge_index[0]
  dst = edge_index[1]
  xp = jnp.pad(x, ((0, NP - N), (0, 0)))
  u, v, Cm, c0 = _tc_prep(xp, W1, W2, b1.reshape(1, D))
  ea = _tc_mm(edge_attr, Cm, c0)
  degp = _sc_indeg(dst)
  dinv, dinv2 = _tc_deg(degp)
  g1 = _sc_scale1(u, dinv)
  tpart = _sc_main(ea, src, dst, v, g1, dinv)
  G = _sc_scale2(tpart, u, dinv, dinv2, b2)
  return _sc_out(ea, src, dst, v, G, dinv2)


# speedup vs baseline: 6.3804x; 1.0386x over previous
import functools

import jax
import jax.numpy as jnp
from jax import lax
from jax.experimental import pallas as pl
from jax.experimental.pallas import tpu as pltpu
from jax.experimental.pallas import tpu_sc as plsc

N = 10000
E = 320000
D = 128
NP = 10240
NC = 2
NS = 16
NW = NC * NS
EPW = E // NW
K = 80
NCH = EPW // K
GRP = 5
GC = NCH // GRP
RPW = NP // NW
RPS = NP // NS

_mesh = plsc.VectorSubcoreMesh(core_axis_name="c", subcore_axis_name="s")
_sc_params = pltpu.CompilerParams(needs_layout_passes=False)


def _floop(n, body):
  lax.fori_loop(0, n, lambda i, c: (body(i), c)[1], 0)


def _bcast16(val_ref, i):
  return plsc.load_gather(val_ref, [jnp.full((16,), i, jnp.int32)])


def _tc_prep_body(x_ref, w1_ref, w2_ref, b1_ref, u_ref, v_ref, c_ref, c0_ref):
  W2 = w2_ref[...]
  xv = x_ref[...]
  A = jnp.dot(w1_ref[0:D, :], W2, preferred_element_type=jnp.float32)
  B = jnp.dot(w1_ref[D:2 * D, :], W2, preferred_element_type=jnp.float32)
  u_ref[...] = jnp.dot(xv, A, preferred_element_type=jnp.float32)
  v_ref[...] = jnp.dot(xv, B, preferred_element_type=jnp.float32)
  c_ref[...] = jnp.dot(w1_ref[2 * D:3 * D, :], W2,
                       preferred_element_type=jnp.float32)
  c0_ref[...] = jnp.dot(b1_ref[...], W2, preferred_element_type=jnp.float32)


def _tc_prep(xp, W1, W2, b1row):
  return pl.pallas_call(
      _tc_prep_body,
      out_shape=[
          jax.ShapeDtypeStruct((NP, D), jnp.float32),
          jax.ShapeDtypeStruct((NP, D), jnp.float32),
          jax.ShapeDtypeStruct((D, D), jnp.float32),
          jax.ShapeDtypeStruct((1, D), jnp.float32),
      ],
  )(xp, W1, W2, b1row)


_MBLK = 2000


def _tc_mm_body(ea_ref, c_ref, c0_ref, o_ref):
  o_ref[...] = jnp.dot(ea_ref[...], c_ref[...],
                       preferred_element_type=jnp.float32) + c0_ref[...]


def _tc_mm(edge_attr, Cm, c0):
  return pl.pallas_call(
      _tc_mm_body,
      grid=(E // _MBLK,),
      in_specs=[
          pl.BlockSpec((_MBLK, D), lambda i: (i, 0)),
          pl.BlockSpec((D, D), lambda i: (0, 0)),
          pl.BlockSpec((1, D), lambda i: (0, 0)),
      ],
      out_specs=pl.BlockSpec((_MBLK, D), lambda i: (i, 0)),
      out_shape=jax.ShapeDtypeStruct((E, D), jnp.float32),
  )(edge_attr, Cm, c0)


def _tc_deg_body(p_ref, dinv_ref, dinv2_ref):
  s = p_ref[0, :] + p_ref[1, :] + 1.0
  dinv_ref[...] = lax.rsqrt(s)
  dinv2_ref[...] = 1.0 / s


def _tc_deg(degp):
  return pl.pallas_call(
      _tc_deg_body,
      out_shape=[
          jax.ShapeDtypeStruct((NP,), jnp.float32),
          jax.ShapeDtypeStruct((NP,), jnp.float32),
      ],
  )(degp)


@functools.partial(
    pl.kernel,
    out_type=jax.ShapeDtypeStruct((NC, NP), jnp.float32),
    mesh=_mesh,
    compiler_params=_sc_params,
    scratch_types=[
        pltpu.VMEM((4, K), jnp.int32),
        pltpu.VMEM((K,), jnp.float32),
        pltpu.VMEM((RPS,), jnp.float32),
        pltpu.VMEM_SHARED((NP,), jnp.float32),
    ] + [pltpu.SemaphoreType.DMA] * 6,
)
def _sc_indeg(dst_hbm, out_hbm, idx_v, ones_v, zb, deg_sh,
              si0, si1, si2, si3, ss0, ss1):
  cid = lax.axis_index("c")
  sid = lax.axis_index("s")
  wid = cid * NS + sid
  sem_i = (si0, si1, si2, si3)
  sem_s = (ss0, ss1)

  def zset(i):
    zb[pl.ds(i * 16, 16)] = jnp.zeros((16,), jnp.float32)
  plsc.parallel_loop(0, RPS // 16)(zset)

  def oset(i):
    ones_v[pl.ds(i * 16, 16)] = jnp.ones((16,), jnp.float32)
  plsc.parallel_loop(0, K // 16)(oset)

  pltpu.sync_copy(zb, deg_sh.at[pl.ds(sid * RPS, RPS)])
  plsc.subcore_barrier()

  def group(grp):
    g0 = grp * GC

    def stage(g):
      base = wid * EPW + (g0 + g) * K
      return pltpu.async_copy(dst_hbm.at[pl.ds(base, K)], idx_v.at[g % 4],
                              sem_i[g % 4])

    st = {0: stage(0), 1: stage(1)}
    sc = {}
    for g in range(GC):
      st[g].wait()
      sc[g] = pltpu.async_copy(ones_v, deg_sh.at[idx_v.at[g % 4]],
                               sem_s[g % 2], add=True)
      if g >= 2:
        sc[g - 2].wait()
      if g + 2 < GC:
        st[g + 2] = stage(g + 2)
    sc[GC - 2].wait()
    sc[GC - 1].wait()
  _floop(GRP, group)

  plsc.subcore_barrier()
  pltpu.sync_copy(deg_sh.at[pl.ds(sid * RPS, RPS)],
                  out_hbm.at[cid, pl.ds(sid * RPS, RPS)])


@functools.partial(
    pl.kernel,
    out_type=jax.ShapeDtypeStruct((NP, D), jnp.float32),
    mesh=_mesh,
    compiler_params=_sc_params,
    scratch_types=[
        pltpu.VMEM((RPW,), jnp.float32),
        pltpu.VMEM((K, D), jnp.float32),
    ],
)
def _sc_scale1(u_hbm, dinv_hbm, g1_hbm, dv, ub):
  cid = lax.axis_index("c")
  sid = lax.axis_index("s")
  wid = cid * NS + sid
  r0 = wid * RPW
  pltpu.sync_copy(dinv_hbm.at[pl.ds(r0, RPW)], dv)
  for c in range(RPW // K):
    pltpu.sync_copy(u_hbm.at[pl.ds(r0 + c * K, K), :], ub)

    def row(i):
      db = _bcast16(dv, c * K + i)
      for f in range(D // 16):
        sl = pl.ds(f * 16, 16)
        ub[i, sl] = ub[i, sl] * db
    plsc.parallel_loop(0, K)(row)
    pltpu.sync_copy(ub, g1_hbm.at[pl.ds(r0 + c * K, K), :])


@functools.partial(
    pl.kernel,
    out_type=jax.ShapeDtypeStruct((NC, NP, D), jnp.float32),
    mesh=_mesh,
    compiler_params=_sc_params,
    scratch_types=[
        pltpu.VMEM((2, K), jnp.int32),
        pltpu.VMEM((2, K), jnp.int32),
        pltpu.VMEM((2, K), jnp.float32),
        pltpu.VMEM((K, D), jnp.float32),
        pltpu.VMEM((2, K, D), jnp.float32),
        pltpu.VMEM((K, D), jnp.float32),
        pltpu.VMEM_SHARED((NP, D), jnp.float32),
    ] + [pltpu.SemaphoreType.DMA] * 11,
)
def _sc_main(ea_hbm, src_hbm, dst_hbm, v_hbm, g1_hbm, dinv_hbm,
             tpart_hbm,
             src_v, dst_v, w_v, sb, vb, gb, tsh,
             si0, si1, se0, se1, sv0, sv1, sg, sw0, sw1, ss0, ss1):
  cid = lax.axis_index("c")
  sid = lax.axis_index("s")
  wid = cid * NS + sid
  sem_i = (si0, si1)
  sem_e = (se0, se1)
  sem_v = (sv0, sv1)
  sem_w = (sw0, sw1)
  sem_s = (ss0, ss1)

  def zrow(i):
    for f in range(D // 16):
      vb[0, i, pl.ds(f * 16, 16)] = jnp.zeros((16,), jnp.float32)
  _floop(K, zrow)
  for q in range(RPS // K):
    pltpu.sync_copy(vb.at[0], tsh.at[pl.ds(sid * RPS + q * K, K), :])
  plsc.subcore_barrier()

  def group(grp):
    g0 = grp * GC

    def stage(g):
      s = g % 2
      base = wid * EPW + (g0 + g) * K
      return (pltpu.async_copy(src_hbm.at[pl.ds(base, K)], src_v.at[s],
                               sem_i[s]),
              pltpu.async_copy(dst_hbm.at[pl.ds(base, K)], dst_v.at[s],
                               sem_i[s]))

    def ea_load(g):
      base = wid * EPW + (g0 + g) * K
      return pltpu.async_copy(ea_hbm.at[pl.ds(base, K), :], sb, se0)

    def gathers(g):
      s = g % 2
      return (pltpu.async_copy(v_hbm.at[dst_v.at[s]], vb.at[s], sem_v[s]),
              pltpu.async_copy(dinv_hbm.at[src_v.at[s]], w_v.at[s],
                               sem_w[s]))

    def g1_gather(g):
      s = g % 2
      return pltpu.async_copy(g1_hbm.at[src_v.at[s]], gb, sg)

    st = {0: stage(0), 1: stage(1)}
    ea = {0: ea_load(0)}
    st[0][0].wait()
    st[0][1].wait()
    ga = {0: gathers(0)}
    gg = {0: g1_gather(0)}
    sc = {}
    for g in range(GC):
      s = g % 2
      if g + 1 < GC:
        if g >= 1:
          sc[g - 1].wait()
        st[g + 1][0].wait()
        st[g + 1][1].wait()
        ga[g + 1] = gathers(g + 1)
      ea[g].wait()
      ga[g][0].wait()
      ga[g][1].wait()
      gg[g].wait()

      def row(i):
        wb = _bcast16(w_v.at[s], i)
        for f in range(D // 16):
          sl = pl.ds(f * 16, 16)
          vb[s, i, sl] = (sb[i, sl] + vb[s, i, sl]) * wb + gb[i, sl]
      plsc.parallel_loop(0, K)(row)

      sc[g] = pltpu.async_copy(vb.at[s], tsh.at[dst_v.at[s]], ss0 if s == 0
                               else ss1, add=True)
      if g + 1 < GC:
        ea[g + 1] = ea_load(g + 1)
        gg[g + 1] = g1_gather(g + 1)
      if g + 2 < GC:
        st[g + 2] = stage(g + 2)
    sc[GC - 2].wait()
    sc[GC - 1].wait()
  _floop(GRP, group)

  plsc.subcore_barrier()
  pltpu.sync_copy(tsh.at[pl.ds(sid * RPS, RPS), :],
                  tpart_hbm.at[cid, pl.ds(sid * RPS, RPS), :])


@functools.partial(
    pl.kernel,
    out_type=jax.ShapeDtypeStruct((NP, D), jnp.float32),
    mesh=_mesh,
    compiler_params=_sc_params,
    scratch_types=[
        pltpu.VMEM((RPW,), jnp.float32),
        pltpu.VMEM((RPW,), jnp.float32),
        pltpu.VMEM((D,), jnp.float32),
        pltpu.VMEM((K, D), jnp.float32),
        pltpu.VMEM((K, D), jnp.float32),
        pltpu.VMEM((K, D), jnp.float32),
    ],
)
def _sc_scale2(tpart_hbm, u_hbm, dinv_hbm, dinv2_hbm, b2_hbm, g_hbm,
               dv, dv2, b2v, t0, t1, ub):
  cid = lax.axis_index("c")
  sid = lax.axis_index("s")
  wid = cid * NS + sid
  r0 = wid * RPW
  pltpu.sync_copy(dinv_hbm.at[pl.ds(r0, RPW)], dv)
  pltpu.sync_copy(dinv2_hbm.at[pl.ds(r0, RPW)], dv2)
  pltpu.sync_copy(b2_hbm, b2v)
  for c in range(RPW // K):
    rb = r0 + c * K
    pltpu.sync_copy(tpart_hbm.at[0, pl.ds(rb, K), :], t0)
    pltpu.sync_copy(tpart_hbm.at[1, pl.ds(rb, K), :], t1)
    pltpu.sync_copy(u_hbm.at[pl.ds(rb, K), :], ub)

    def row(i):
      db = _bcast16(dv, c * K + i)
      d2b = _bcast16(dv2, c * K + i)
      for f in range(D // 16):
        sl = pl.ds(f * 16, 16)
        t0[i, sl] = (t0[i, sl] + t1[i, sl]) * db + ub[i, sl] * d2b + b2v[sl]
    plsc.parallel_loop(0, K)(row)
    pltpu.sync_copy(t0, g_hbm.at[pl.ds(rb, K), :])


@functools.partial(
    pl.kernel,
    out_type=jax.ShapeDtypeStruct((E, D), jnp.float32),
    mesh=_mesh,
    compiler_params=_sc_params,
    scratch_types=[
        pltpu.VMEM((2, K), jnp.int32),
        pltpu.VMEM((2, K), jnp.int32),
        pltpu.VMEM((2, K), jnp.float32),
        pltpu.VMEM((2, K, D), jnp.float32),
        pltpu.VMEM((2, K, D), jnp.float32),
        pltpu.VMEM((2, K, D), jnp.float32),
        pltpu.VMEM((2, K, D), jnp.float32),
    ] + [pltpu.SemaphoreType.DMA] * 12,
)
def _sc_out(ea_hbm, src_hbm, dst_hbm, v_hbm, g_hbm, dinv2_hbm,
            out_hbm, src_v, dst_v, w2_v, sb, vb, gb, ob,
            si0, si1, se0, se1, sv0, sv1, sg0, sg1, sw0, sw1, so0, so1):
  cid = lax.axis_index("c")
  sid = lax.axis_index("s")
  wid = cid * NS + sid
  sem_i = (si0, si1)
  sem_e = (se0, se1)
  sem_v = (sv0, sv1)
  sem_g = (sg0, sg1)
  sem_w = (sw0, sw1)
  sem_o = (so0, so1)

  def group(grp):
    g0 = grp * GC

    def stage(g):
      s = g % 2
      base = wid * EPW + (g0 + g) * K
      return (pltpu.async_copy(src_hbm.at[pl.ds(base, K)], src_v.at[s],
                               sem_i[s]),
              pltpu.async_copy(dst_hbm.at[pl.ds(base, K)], dst_v.at[s],
                               sem_i[s]),
              pltpu.async_copy(ea_hbm.at[pl.ds(base, K), :], sb.at[s],
                               sem_e[s]))

    def gathers(g):
      s = g % 2
      return (pltpu.async_copy(v_hbm.at[dst_v.at[s]], vb.at[s], sem_v[s]),
              pltpu.async_copy(g_hbm.at[src_v.at[s]], gb.at[s], sem_g[s]),
              pltpu.async_copy(dinv2_hbm.at[src_v.at[s]], w2_v.at[s],
                               sem_w[s]))

    st = {0: stage(0), 1: stage(1)}
    st[0][0].wait()
    st[0][1].wait()
    ga = {0: gathers(0)}
    wr = {}
    for g in range(GC):
      s = g % 2
      if g >= 2:
        wr[g - 2].wait()
      if g + 1 < GC:
        st[g + 1][0].wait()
        st[g + 1][1].wait()
        ga[g + 1] = gathers(g + 1)
      st[g][2].wait()
      ga[g][0].wait()
      ga[g][1].wait()
      ga[g][2].wait()

      def row(i):
        w2b = _bcast16(w2_v.at[s], i)
        for f in range(D // 16):
          sl = pl.ds(f * 16, 16)
          o = (sb[s, i, sl] + vb[s, i, sl]) * w2b + gb[s, i, sl]
          ob[s, i, sl] = jnp.maximum(o, 0.01 * o)
      plsc.parallel_loop(0, K)(row)

      base = wid * EPW + (g0 + g) * K
      wr[g] = pltpu.async_copy(ob.at[s], out_hbm.at[pl.ds(base, K), :],
                               sem_o[s])
      if g + 2 < GC:
        st[g + 2] = stage(g + 2)
    wr[GC - 2].wait()
    wr[GC - 1].wait()
  _floop(GRP, group)


@jax.jit
def kernel(x, edge_index, batch, edge_attr, W1, b1, W2, b2):
  src = edge_index[0]
  dst = edge_index[1]
  xp = jnp.pad(x, ((0, NP - N), (0, 0)))
  u, v, Cm, c0 = _tc_prep(xp, W1, W2, b1.reshape(1, D))
  ea = _tc_mm(edge_attr, Cm, c0)
  degp = _sc_indeg(dst)
  dinv, dinv2 = _tc_deg(degp)
  g1 = _sc_scale1(u, dinv)
  tpart = _sc_main(ea, src, dst, v, g1, dinv)
  G = _sc_scale2(tpart, u, dinv, dinv2, b2)
  return _sc_out(ea, src, dst, v, G, dinv2)
